# Initial kernel scaffold; baseline (speedup 1.0000x reference)
import jax
import jax.numpy as jnp
from jax.experimental import pallas as pl


def kernel(local_entity, q2e_adj_mat, kb_fact_rel, query_text, head_idx, tail_idx, entity_table, relation_table, word_table, ent_W, ent_b, rel_W, rel_b, lstm_Wih, lstm_Whh, lstm_b, q2e_W, q2e_b, kb_head_W, kb_head_b, kb_tail_W, kb_tail_b, kb_self_W, kb_self_b, e2e_W, e2e_b, score_W, score_b):
    raise NotImplementedError("write your pallas kernel here")


# speedup vs baseline: 5.3455x; 5.3455x over previous
import functools

import jax
import jax.numpy as jnp
from jax import lax
from jax.experimental import pallas as pl
from jax.experimental.pallas import tpu as pltpu
from jax.experimental.pallas import tpu_sc as plsc

B, E, F, Q, D = 8, 2000, 8000, 16, 100
EP, FP, DP = 2048, 8192, 128
EB, FB = B * EP, B * FP
NE, NR, NW = 200000, 500, 50000
NL = 3
NC, NS = 2, 16
VERY_NEG = -1e11
PL = 0.8
FS = 3.0
_f32 = jnp.float32


def _sc_gather(table, idx, CH):
    N = idx.shape[0]
    DW = table.shape[1]
    n = N // (NC * NS)
    nch = n // CH
    mesh = plsc.VectorSubcoreMesh(core_axis_name="c", subcore_axis_name="s")

    @functools.partial(
        pl.kernel, mesh=mesh,
        out_type=jax.ShapeDtypeStruct((N, DW), _f32),
        scratch_types=[pltpu.VMEM((n,), jnp.int32),
                       pltpu.VMEM((CH, DW), _f32),
                       pltpu.SemaphoreType.DMA])
    def k(tab, ix, out, idx_v, rows_v, sem):
        wid = lax.axis_index("s") * NC + lax.axis_index("c")
        base = wid * n
        pltpu.sync_copy(ix.at[pl.ds(base, n)], idx_v)

        def body(j, carry):
            pltpu.async_copy(tab.at[idx_v.at[pl.ds(j * CH, CH)]], rows_v,
                             sem).wait()
            pltpu.sync_copy(rows_v, out.at[pl.ds(base + j * CH, CH)])
            return carry

        lax.fori_loop(0, nch, body, 0)

    return k(table, idx)


def _sc_scatter(zeros, msgs, idx2d):
    EG, CH = 4 * EP, 128
    mesh = plsc.VectorSubcoreMesh(core_axis_name="c", subcore_axis_name="s")

    @functools.partial(
        pl.kernel, mesh=mesh,
        out_type=jax.ShapeDtypeStruct((2 * EG, DP), _f32),
        scratch_types=[pltpu.VMEM((CH,), jnp.int32),
                       pltpu.VMEM((CH, DP), _f32),
                       pltpu.VMEM_SHARED((EG, DP), _f32)])
    def k(zer, ms, ix2, out, idx_v, buf_v, acc):
        c = lax.axis_index("c")
        s = lax.axis_index("s")
        pltpu.sync_copy(zer, acc.at[pl.ds(s * 512, 512)])
        plsc.subcore_barrier()

        def body(j, carry):
            chunk = c * 256 + s * 16 + j
            pltpu.sync_copy(ms.at[pl.ds(chunk * CH, CH)], buf_v)
            pltpu.sync_copy(ix2.at[chunk], idx_v)
            pltpu.sync_copy(buf_v, acc.at[idx_v], add=True)
            return carry

        lax.fori_loop(0, 16, body, 0)
        plsc.subcore_barrier()
        pltpu.sync_copy(acc.at[pl.ds(s * 512, 512)],
                        out.at[pl.ds(c * EG + s * 512, 512)])

    return k(zeros, msgs, idx2d)


def _sc_scalar_scatter(idx, val):
    EG, n = 4 * EP, FB // (NC * NS)
    mesh = plsc.VectorSubcoreMesh(core_axis_name="c", subcore_axis_name="s")

    @functools.partial(
        pl.kernel, mesh=mesh,
        compiler_params=pltpu.CompilerParams(needs_layout_passes=False),
        out_type=jax.ShapeDtypeStruct((2 * NS, EG), _f32),
        scratch_types=[pltpu.VMEM((n,), jnp.int32),
                       pltpu.VMEM((n,), _f32),
                       pltpu.VMEM((EG,), _f32)])
    def k(ix, vl, out, idx_v, val_v, acc):
        c = lax.axis_index("c")
        s = lax.axis_index("s")
        base = (c * NS + s) * n

        def zbody(i, carry):
            acc[pl.ds(i * 16, 16)] = jnp.zeros((16,), _f32)
            return carry

        lax.fori_loop(0, EG // 16, zbody, 0)
        pltpu.sync_copy(ix.at[pl.ds(base, n)], idx_v)
        pltpu.sync_copy(vl.at[pl.ds(base, n)], val_v)

        def body(j, carry):
            iv = idx_v[pl.ds(j * 16, 16)]
            vv = jnp.maximum(val_v[pl.ds(j * 16, 16)], 0.0)
            plsc.addupdate_scatter(acc, [iv], vv)
            return carry

        lax.fori_loop(0, n // 16, body, 0)
        pltpu.sync_copy(acc, out.at[c * NS + s])

    return k(idx, val)


def _dot(a, b):
    return jnp.dot(a, b, preferred_element_type=_f32)


def _sig(x):
    return 1.0 / (1.0 + jnp.exp(-x))


def _proj_body(x_ref, w_ref, b_ref, o_ref):
    o_ref[...] = _dot(x_ref[...], w_ref[...]) + b_ref[...]


def _lstm_body(wr_ref, wih_ref, whh_ref, b_ref, hs_ref, node_ref):
    def step(q, carry):
        h, c = carry
        x = wr_ref[pl.ds(q * 8, 8), :]
        z = _dot(x, wih_ref[...]) + _dot(h, whh_ref[...]) + b_ref[...]
        i = _sig(z[:, 0:DP])
        f = _sig(z[:, DP:2 * DP])
        g = jnp.tanh(z[:, 2 * DP:3 * DP])
        o = _sig(z[:, 3 * DP:4 * DP])
        c = f * c + i * g
        h = o * jnp.tanh(c)
        hs_ref[pl.ds(q, 1)] = h[None]
        return (h, c)

    z0 = jnp.zeros((8, DP), _f32)
    h, _ = lax.fori_loop(0, Q, step, (z0, z0))
    node_ref[...] = h


def _att_body(hs_ref, fact_ref, qmt_ref, o_ref):
    h = hs_ref[...].reshape(Q, DP)
    pen_row = qmt_ref[...].reshape(1, Q)
    pen = lax.dot_general(jnp.eye(Q, dtype=_f32), pen_row,
                          (((1,), (1,)), ((), ())),
                          preferred_element_type=_f32)
    sim = lax.dot_general(h, fact_ref[...],
                          (((1,), (1,)), ((), ())),
                          preferred_element_type=_f32) * (1.0 / 10.0)
    masked = sim + (1.0 - pen) * VERY_NEG
    mx = jnp.max(masked, axis=0, keepdims=True)
    ex = jnp.exp(masked - mx)
    p = ex / jnp.sum(ex, axis=0, keepdims=True)
    wf = jnp.sum(p * sim, axis=0, keepdims=True)
    fmask = lax.broadcasted_iota(jnp.int32, (1, FP), 1) < F
    wmax = jnp.max(jnp.where(fmask, wf, -1e30))
    wt = jnp.exp(wf - wmax)
    o_ref[...] = jnp.where(fmask, wt, -1.0)[None]


def _e0_body(emb_ref, part_ref, pr_ref, w_ref, b_ref, hh_ref, e2f_ref):
    ones = jnp.ones((Q, 1), _f32)
    psum = lax.dot_general(part_ref[...], ones,
                           (((0,), (0,)), ((), ())),
                           preferred_element_type=_f32)
    e2f = jnp.maximum(psum, 1e-10)
    ratio = pr_ref[...] / e2f
    hh = _dot(emb_ref[...], w_ref[...]) + b_ref[...]
    colm = lax.broadcasted_iota(jnp.int32, (EP, DP), 1) == D
    hh_ref[...] = jnp.where(colm, ratio, hh)
    e2f_ref[...] = e2f


def _col(x):
    m = lax.broadcasted_iota(jnp.int32, x.shape, 1) == D
    return jnp.sum(jnp.where(m, x, 0.0), axis=1, keepdims=True)


def _payload_body(g_ref, fact_ref, wt_ref, sw_ref, sb_ref, tw_ref, tb_ref,
                  o_ref):
    wt = wt_ref[...]
    valid = wt >= 0.0
    s = jnp.maximum(wt, 0.0) * _col(g_ref[...])
    e2f = jnp.maximum(_dot(fact_ref[...], sw_ref[...]) + sb_ref[...]
                      + g_ref[...], 0.0)
    th = _dot(e2f, tw_ref[...])
    pay = jnp.where(valid, th * s + tb_ref[...], 0.0)
    colm = lax.broadcasted_iota(jnp.int32, (pay.shape[0], DP), 1) == D
    o_ref[...] = jnp.where(colm, s, pay)


def _combine_body(agg_ref, emb_ref, pr_ref, e2f_ref, node_ref, qw_ref,
                  qb_ref, sw_ref, sb_ref, w1_ref, w2_ref, w3_ref, eb_ref,
                  hw_ref, hb_ref, emb_o, pr_o, hh_o):
    nb = node_ref[...].reshape(1, DP)
    q2e = _dot(_dot(nb, qw_ref[...]) + qb_ref[...], w2_ref[...])
    f2e = jnp.maximum(_dot(emb_ref[...], sw_ref[...]) + sb_ref[...]
                      + agg_ref[...], 0.0)
    z = (_dot(emb_ref[...], w1_ref[...]) + q2e
         + _dot(f2e, w3_ref[...]) * FS + eb_ref[...])
    new_emb = jnp.maximum(z, 0.0)
    new_pr = PL * _col(agg_ref[...]) + (1.0 - PL) * pr_ref[...]
    hh = _dot(new_emb, hw_ref[...]) + hb_ref[...]
    ratio = new_pr / e2f_ref[...]
    colm = lax.broadcasted_iota(jnp.int32, (EP, DP), 1) == D
    emb_o[...] = new_emb
    pr_o[...] = new_pr
    hh_o[...] = jnp.where(colm, ratio, hh)


def _final_body(agg_ref, emb_ref, node_ref, le_ref, qw_ref, qb_ref, sw_ref,
                sb_ref, w1_ref, w2_ref, w3_ref, eb_ref, scw_ref, scb_ref,
                o_ref):
    nb = node_ref[...].reshape(1, DP)
    q2e = _dot(_dot(nb, qw_ref[...]) + qb_ref[...], w2_ref[...])
    f2e = jnp.maximum(_dot(emb_ref[...], sw_ref[...]) + sb_ref[...]
                      + agg_ref[...], 0.0)
    z = (_dot(emb_ref[...], w1_ref[...]) + q2e
         + _dot(f2e, w3_ref[...]) * FS + eb_ref[...])
    new_emb = jnp.maximum(z, 0.0)
    sc = lax.dot_general(scw_ref[...], new_emb,
                         (((1,), (1,)), ((), ())),
                         preferred_element_type=_f32) + scb_ref[...]
    mask = (le_ref[...].reshape(1, EP) != NE).astype(_f32)
    o_ref[...] = (sc + (1.0 - mask) * VERY_NEG)[None]


def _wspec(shape):
    nd = len(shape)
    return pl.BlockSpec(shape, lambda *a, _nd=nd: (0,) * _nd)


def _tc_proj(x, wt, b, bm):
    n, kk = x.shape
    return pl.pallas_call(
        _proj_body, grid=(n // bm,),
        in_specs=[pl.BlockSpec((bm, kk), lambda i: (i, 0)),
                  _wspec(wt.shape), _wspec(b.shape)],
        out_specs=pl.BlockSpec((bm, DP), lambda i: (i, 0)),
        out_shape=jax.ShapeDtypeStruct((n, DP), _f32))(x, wt, b)


def _tc_lstm(word_rows, wih, whh, b):
    return pl.pallas_call(
        _lstm_body,
        in_specs=[_wspec(word_rows.shape), _wspec(wih.shape),
                  _wspec(whh.shape), _wspec(b.shape)],
        out_specs=[_wspec((Q, 8, DP)), _wspec((8, DP))],
        out_shape=[jax.ShapeDtypeStruct((Q, 8, DP), _f32),
                   jax.ShapeDtypeStruct((8, DP), _f32)])(
                       word_rows, wih, whh, b)


def _tc_att(hs_t, fact_emb, qm3):
    return pl.pallas_call(
        _att_body, grid=(B,),
        in_specs=[pl.BlockSpec((1, Q, DP), lambda b: (b, 0, 0)),
                  pl.BlockSpec((FP, DP), lambda b: (b, 0)),
                  pl.BlockSpec((1, 1, Q), lambda b: (b, 0, 0))],
        out_specs=pl.BlockSpec((1, 1, FP), lambda b: (b, 0, 0)),
        out_shape=jax.ShapeDtypeStruct((B, 1, FP), _f32))(hs_t, fact_emb, qm3)


def _tc_e0(ent_emb, partials, pr0, hw, hb):
    return pl.pallas_call(
        _e0_body, grid=(B,),
        in_specs=[pl.BlockSpec((EP, DP), lambda b: (b, 0)),
                  pl.BlockSpec((NS, EP), lambda b: (b // 4, b % 4)),
                  pl.BlockSpec((EP, 1), lambda b: (b, 0)),
                  _wspec((DP, DP)), _wspec((1, DP))],
        out_specs=[pl.BlockSpec((EP, DP), lambda b: (b, 0)),
                   pl.BlockSpec((EP, 1), lambda b: (b, 0))],
        out_shape=[jax.ShapeDtypeStruct((EB, DP), _f32),
                   jax.ShapeDtypeStruct((EB, 1), _f32)])(
                       ent_emb, partials, pr0, hw, hb)


def _tc_payload(g, fact_emb, wt_col, sw, sb, tw, tb):
    bm = 4096
    return pl.pallas_call(
        _payload_body, grid=(FB // bm,),
        in_specs=[pl.BlockSpec((bm, DP), lambda i: (i, 0)),
                  pl.BlockSpec((bm, DP), lambda i: (i, 0)),
                  pl.BlockSpec((bm, 1), lambda i: (i, 0)),
                  _wspec((DP, DP)), _wspec((1, DP)),
                  _wspec((DP, DP)), _wspec((1, DP))],
        out_specs=pl.BlockSpec((bm, DP), lambda i: (i, 0)),
        out_shape=jax.ShapeDtypeStruct((FB, DP), _f32))(
            g, fact_emb, wt_col, sw, sb, tw, tb)


def _tc_combine(agg, ent_emb, pr, e2f, node, qw, qb, sw, sb, w1, w2, w3,
                eb, hw, hb):
    eblk = lambda b: (b, 0)
    return pl.pallas_call(
        _combine_body, grid=(B,),
        in_specs=[pl.BlockSpec((EP, DP), eblk), pl.BlockSpec((EP, DP), eblk),
                  pl.BlockSpec((EP, 1), eblk), pl.BlockSpec((EP, 1), eblk),
                  pl.BlockSpec((1, 1, DP), lambda b: (b, 0, 0)),
                  _wspec((DP, DP)), _wspec((1, DP)),
                  _wspec((DP, DP)), _wspec((1, DP)), _wspec((DP, DP)),
                  _wspec((DP, DP)), _wspec((DP, DP)), _wspec((1, DP)),
                  _wspec((DP, DP)), _wspec((1, DP))],
        out_specs=[pl.BlockSpec((EP, DP), eblk), pl.BlockSpec((EP, 1), eblk),
                   pl.BlockSpec((EP, DP), eblk)],
        out_shape=[jax.ShapeDtypeStruct((EB, DP), _f32),
                   jax.ShapeDtypeStruct((EB, 1), _f32),
                   jax.ShapeDtypeStruct((EB, DP), _f32)])(
                       agg, ent_emb, pr, e2f, node, qw, qb, sw, sb,
                       w1, w2, w3, eb, hw, hb)


def _tc_final(agg, ent_emb, node, le3, qw, qb, sw, sb, w1, w2, w3, eb,
              scw, scb):
    eblk = lambda b: (b, 0)
    return pl.pallas_call(
        _final_body, grid=(B,),
        in_specs=[pl.BlockSpec((EP, DP), eblk), pl.BlockSpec((EP, DP), eblk),
                  pl.BlockSpec((1, 1, DP), lambda b: (b, 0, 0)),
                  pl.BlockSpec((1, 1, EP), lambda b: (b, 0, 0)),
                  _wspec((DP, DP)), _wspec((1, DP)),
                  _wspec((DP, DP)), _wspec((1, DP)), _wspec((DP, DP)),
                  _wspec((DP, DP)), _wspec((DP, DP)), _wspec((1, DP)),
                  _wspec((1, DP)), _wspec((1, 1))],
        out_specs=pl.BlockSpec((1, 1, EP), lambda b: (b, 0, 0)),
        out_shape=jax.ShapeDtypeStruct((B, 1, EP), _f32))(
            agg, ent_emb, node, le3, qw, qb, sw, sb, w1, w2, w3, eb,
            scw, scb)


def _padT(w):
    o, i = w.shape
    return jnp.zeros((((i + 127) // 128) * 128, DP), _f32).at[:i, :o].set(w.T)


def _padb(b):
    return jnp.zeros((1, DP), _f32).at[0, :b.shape[0]].set(b)


def kernel(local_entity, q2e_adj_mat, kb_fact_rel, query_text, head_idx,
           tail_idx, entity_table, relation_table, word_table, ent_W, ent_b,
           rel_W, rel_b, lstm_Wih, lstm_Whh, lstm_b, q2e_W, q2e_b,
           kb_head_W, kb_head_b, kb_tail_W, kb_tail_b, kb_self_W, kb_self_b,
           e2e_W, e2e_b, score_W, score_b):
    local_entity = local_entity.astype(jnp.int32)
    kb_fact_rel = kb_fact_rel.astype(jnp.int32)
    query_text = query_text.astype(jnp.int32)
    head_idx = head_idx.astype(jnp.int32)
    tail_idx = tail_idx.astype(jnp.int32)

    ent_tab = jnp.pad(entity_table, ((0, 0), (0, DP - D)))
    word_tab = jnp.pad(word_table, ((0, 0), (0, DP - D)))
    rel_tab = jnp.pad(relation_table, ((0, 0), (0, 256 - 2 * D)))

    entWt, entb = _padT(ent_W), _padb(ent_b)
    relWt, relb = _padT(rel_W), _padb(rel_b)
    wih = jnp.zeros((DP, 4, DP), _f32).at[:D, :, :D].set(
        lstm_Wih.reshape(4, D, D).transpose(2, 0, 1)).reshape(DP, 4 * DP)
    whh = jnp.zeros((DP, 4, DP), _f32).at[:D, :, :D].set(
        lstm_Whh.reshape(4, D, D).transpose(2, 0, 1)).reshape(DP, 4 * DP)
    lstmb = jnp.zeros((4, DP), _f32).at[:, :D].set(
        lstm_b.reshape(4, D)).reshape(1, 4 * DP)
    qWt = [_padT(q2e_W[i]) for i in range(NL)]
    qb = [_padb(q2e_b[i]) for i in range(NL)]
    hWt = [_padT(kb_head_W[i]) for i in range(NL)]
    hb = [_padb(kb_head_b[i]) for i in range(NL)]
    tWt = [_padT(kb_tail_W[i]) for i in range(NL)]
    tb = [_padb(kb_tail_b[i]) for i in range(NL)]
    sWt = [_padT(kb_self_W[i]) for i in range(NL)]
    sb = [_padb(kb_self_b[i]) for i in range(NL)]
    e1t = [_padT(e2e_W[i][:, 0 * D:1 * D]) for i in range(NL)]
    e2t = [_padT(e2e_W[i][:, 1 * D:2 * D]) for i in range(NL)]
    e3t = [_padT(e2e_W[i][:, 2 * D:3 * D]) for i in range(NL)]
    eb = [_padb(e2e_b[i]) for i in range(NL)]
    scw = jnp.zeros((1, DP), _f32).at[0, :D].set(score_W[0])
    scb = score_b.reshape(1, 1)

    le_pad = jnp.pad(local_entity, ((0, 0), (0, EP - E)), constant_values=NE)
    ent_gidx = le_pad.reshape(-1)
    word_idx = jnp.pad(query_text.T.reshape(-1), (0, 128),
                       constant_values=NW)
    rel_idx = jnp.pad(kb_fact_rel, ((0, 0), (0, FP - F)),
                      constant_values=NR).reshape(-1)
    head_pad = jnp.pad(head_idx, ((0, 0), (0, FP - F)))
    tail_pad = jnp.pad(tail_idx, ((0, 0), (0, FP - F)))
    boff = (jnp.arange(B, dtype=jnp.int32) * EP)[:, None]
    goff = ((jnp.arange(B, dtype=jnp.int32) % 4) * EP)[:, None]
    head_gidx = (head_pad + boff).reshape(-1)
    head_lidx = (head_pad + goff).reshape(-1)
    tail_lidx2d = (tail_pad + goff).reshape(-1).reshape(FB // 128, 128)
    pr0 = jnp.pad(q2e_adj_mat[:, :, 0], ((0, 0), (0, EP - E))).reshape(EB, 1)
    qmt = (query_text.T != NW).astype(_f32)
    le3 = le_pad.reshape(B, 1, EP)
    zeros512 = jnp.zeros((512, DP), _f32)

    ent_rows = _sc_gather(ent_tab, ent_gidx, 512)
    word_rows = _sc_gather(word_tab, word_idx, 8)
    rel_rows = _sc_gather(rel_tab, rel_idx, 256)
    ent_emb = _tc_proj(ent_rows, entWt, entb, 2048)
    fact_emb = _tc_proj(rel_rows, relWt, relb, 2048)

    hs, node = _tc_lstm(word_rows, wih, whh, lstmb)
    hs_t = jnp.transpose(hs, (1, 0, 2))
    node3 = node.reshape(B, 1, DP)
    wt = _tc_att(hs_t, fact_emb, qmt.T.reshape(B, 1, Q))
    wt_flat = wt.reshape(-1)
    wt_col = wt.reshape(FB, 1)

    partials = _sc_scalar_scatter(head_lidx, wt_flat)
    hh, e2f = _tc_e0(ent_emb, partials, pr0, hWt[0], hb[0])
    pr = pr0

    for i in range(NL):
        g = _sc_gather(hh, head_gidx, 512)
        payload = _tc_payload(g, fact_emb, wt_col, sWt[i], sb[i],
                              tWt[i], tb[i])
        agg = _sc_scatter(zeros512, payload, tail_lidx2d)
        if i < NL - 1:
            ent_emb, pr, hh = _tc_combine(
                agg, ent_emb, pr, e2f, node3, qWt[i], qb[i], sWt[i], sb[i],
                e1t[i], e2t[i], e3t[i], eb[i], hWt[i + 1], hb[i + 1])
        else:
            score = _tc_final(
                agg, ent_emb, node3, le3, qWt[i], qb[i], sWt[i], sb[i],
                e1t[i], e2t[i], e3t[i], eb[i], scw, scb)

    return score.reshape(B, EP)[:, :E]


# speedup vs baseline: 7.6053x; 1.4227x over previous
import functools

import jax
import jax.numpy as jnp
from jax import lax
from jax.experimental import pallas as pl
from jax.experimental.pallas import tpu as pltpu
from jax.experimental.pallas import tpu_sc as plsc

B, E, F, Q, D = 8, 2000, 8000, 16, 100
EP, FP, DP = 2048, 8192, 128
EB, FB = B * EP, B * FP
NE, NR, NW = 200000, 500, 50000
NL = 3
NC, NS = 2, 16
VERY_NEG = -1e11
PL = 0.8
FS = 3.0
_f32 = jnp.float32


def _sc_gather(table, idx, CH):
    N = idx.shape[0]
    DW = table.shape[1]
    n = N // (NC * NS)
    nch = n // CH
    mesh = plsc.VectorSubcoreMesh(core_axis_name="c", subcore_axis_name="s")

    @functools.partial(
        pl.kernel, mesh=mesh,
        out_type=jax.ShapeDtypeStruct((N, DW), _f32),
        scratch_types=[pltpu.VMEM((n,), jnp.int32),
                       pltpu.VMEM((CH, DW), _f32),
                       pltpu.SemaphoreType.DMA])
    def k(tab, ix, out, idx_v, rows_v, sem):
        wid = lax.axis_index("s") * NC + lax.axis_index("c")
        base = wid * n
        pltpu.sync_copy(ix.at[pl.ds(base, n)], idx_v)

        def body(j, carry):
            pltpu.async_copy(tab.at[idx_v.at[pl.ds(j * CH, CH)]], rows_v,
                             sem).wait()
            pltpu.sync_copy(rows_v, out.at[pl.ds(base + j * CH, CH)])
            return carry

        lax.fori_loop(0, nch, body, 0)

    return k(table, idx)


def _sc_scatter(zeros, msgs, idx2d):
    EG, CH = 4 * EP, 128
    mesh = plsc.VectorSubcoreMesh(core_axis_name="c", subcore_axis_name="s")

    @functools.partial(
        pl.kernel, mesh=mesh,
        out_type=jax.ShapeDtypeStruct((2 * EG, DP), _f32),
        scratch_types=[pltpu.VMEM((CH,), jnp.int32),
                       pltpu.VMEM((CH, DP), _f32),
                       pltpu.VMEM_SHARED((EG, DP), _f32)])
    def k(zer, ms, ix2, out, idx_v, buf_v, acc):
        c = lax.axis_index("c")
        s = lax.axis_index("s")
        pltpu.sync_copy(zer, acc.at[pl.ds(s * 512, 512)])
        plsc.subcore_barrier()

        def body(j, carry):
            chunk = c * 256 + s * 16 + j
            pltpu.sync_copy(ms.at[pl.ds(chunk * CH, CH)], buf_v)
            pltpu.sync_copy(ix2.at[chunk], idx_v)
            pltpu.sync_copy(buf_v, acc.at[idx_v], add=True)
            return carry

        lax.fori_loop(0, 16, body, 0)
        plsc.subcore_barrier()
        pltpu.sync_copy(acc.at[pl.ds(s * 512, 512)],
                        out.at[pl.ds(c * EG + s * 512, 512)])

    return k(zeros, msgs, idx2d)


def _sc_scalar_scatter(idx, val):
    EG, n = 4 * EP, FB // (NC * NS)
    mesh = plsc.VectorSubcoreMesh(core_axis_name="c", subcore_axis_name="s")

    @functools.partial(
        pl.kernel, mesh=mesh,
        compiler_params=pltpu.CompilerParams(needs_layout_passes=False),
        out_type=jax.ShapeDtypeStruct((2 * NS, EG), _f32),
        scratch_types=[pltpu.VMEM((n,), jnp.int32),
                       pltpu.VMEM((n,), _f32),
                       pltpu.VMEM((EG,), _f32)])
    def k(ix, vl, out, idx_v, val_v, acc):
        c = lax.axis_index("c")
        s = lax.axis_index("s")
        base = (c * NS + s) * n

        def zbody(i, carry):
            acc[pl.ds(i * 16, 16)] = jnp.zeros((16,), _f32)
            return carry

        lax.fori_loop(0, EG // 16, zbody, 0)
        pltpu.sync_copy(ix.at[pl.ds(base, n)], idx_v)
        pltpu.sync_copy(vl.at[pl.ds(base, n)], val_v)

        def body(j, carry):
            iv = idx_v[pl.ds(j * 16, 16)]
            vv = jnp.maximum(val_v[pl.ds(j * 16, 16)], 0.0)
            plsc.addupdate_scatter(acc, [iv], vv)
            return carry

        lax.fori_loop(0, n // 16, body, 0)
        pltpu.sync_copy(acc, out.at[c * NS + s])

    return k(idx, val)


def _dot(a, b):
    return jnp.dot(a, b, preferred_element_type=_f32)


def _sig(x):
    return 1.0 / (1.0 + jnp.exp(-x))


def _proj_body(x_ref, w_ref, b_ref, o_ref):
    o_ref[...] = _dot(x_ref[...], w_ref[...]) + b_ref[...]


def _pad_body(x_ref, o_ref):
    o_ref[...] = jnp.pad(x_ref[...], ((0, 0), (0, DP - D)))


def _tc_pad(x, bm):
    n = x.shape[0]
    return pl.pallas_call(
        _pad_body, grid=(pl.cdiv(n, bm),),
        in_specs=[pl.BlockSpec((bm, D), lambda i: (i, 0))],
        out_specs=pl.BlockSpec((bm, DP), lambda i: (i, 0)),
        out_shape=jax.ShapeDtypeStruct((n, DP), _f32))(x)


def _lstm_body(wr_ref, wih_ref, whh_ref, b_ref, hs_ref, node_ref):
    def step(q, carry):
        h, c = carry
        x = wr_ref[pl.ds(q * 8, 8), :]
        z = _dot(x, wih_ref[...]) + _dot(h, whh_ref[...]) + b_ref[...]
        i = _sig(z[:, 0:DP])
        f = _sig(z[:, DP:2 * DP])
        g = jnp.tanh(z[:, 2 * DP:3 * DP])
        o = _sig(z[:, 3 * DP:4 * DP])
        c = f * c + i * g
        h = o * jnp.tanh(c)
        hs_ref[pl.ds(q, 1)] = h[None]
        return (h, c)

    z0 = jnp.zeros((8, DP), _f32)
    h, _ = lax.fori_loop(0, Q, step, (z0, z0))
    node_ref[...] = h


def _att_body(hs_ref, fact_ref, qmt_ref, o_ref):
    h = hs_ref[...].reshape(Q, DP)
    pen_row = qmt_ref[...].reshape(1, Q)
    pen = lax.dot_general(jnp.eye(Q, dtype=_f32), pen_row,
                          (((1,), (1,)), ((), ())),
                          preferred_element_type=_f32)
    sim = lax.dot_general(h, fact_ref[...],
                          (((1,), (1,)), ((), ())),
                          preferred_element_type=_f32) * (1.0 / 10.0)
    masked = sim + (1.0 - pen) * VERY_NEG
    mx = jnp.max(masked, axis=0, keepdims=True)
    ex = jnp.exp(masked - mx)
    p = ex / jnp.sum(ex, axis=0, keepdims=True)
    wf = jnp.sum(p * sim, axis=0, keepdims=True)
    fmask = lax.broadcasted_iota(jnp.int32, (1, FP), 1) < F
    wmax = jnp.max(jnp.where(fmask, wf, -1e30))
    wt = jnp.exp(wf - wmax)
    o_ref[...] = jnp.where(fmask, wt, -1.0)[None]


def _e0_body(emb_ref, part_ref, pr_ref, w_ref, b_ref, hh_ref, e2f_ref):
    ones = jnp.ones((Q, 1), _f32)
    psum = lax.dot_general(part_ref[...], ones,
                           (((0,), (0,)), ((), ())),
                           preferred_element_type=_f32)
    e2f = jnp.maximum(psum, 1e-10)
    ratio = pr_ref[...] / e2f
    hh = _dot(emb_ref[...], w_ref[...]) + b_ref[...]
    colm = lax.broadcasted_iota(jnp.int32, (EP, DP), 1) == D
    hh_ref[...] = jnp.where(colm, ratio, hh)
    e2f_ref[...] = e2f


def _col(x):
    m = lax.broadcasted_iota(jnp.int32, x.shape, 1) == D
    return jnp.sum(jnp.where(m, x, 0.0), axis=1, keepdims=True)


def _payload_body(g_ref, fact_ref, wt_ref, sw_ref, sb_ref, tw_ref, tb_ref,
                  o_ref):
    wt = wt_ref[...]
    valid = wt >= 0.0
    s = jnp.maximum(wt, 0.0) * _col(g_ref[...])
    e2f = jnp.maximum(_dot(fact_ref[...], sw_ref[...]) + sb_ref[...]
                      + g_ref[...], 0.0)
    th = _dot(e2f, tw_ref[...])
    pay = jnp.where(valid, th * s + tb_ref[...], 0.0)
    colm = lax.broadcasted_iota(jnp.int32, (pay.shape[0], DP), 1) == D
    o_ref[...] = jnp.where(colm, s, pay)


def _combine_body(agg_ref, emb_ref, pr_ref, e2f_ref, node_ref, qw_ref,
                  qb_ref, sw_ref, sb_ref, w1_ref, w2_ref, w3_ref, eb_ref,
                  hw_ref, hb_ref, emb_o, pr_o, hh_o):
    nb = node_ref[...].reshape(1, DP)
    q2e = _dot(_dot(nb, qw_ref[...]) + qb_ref[...], w2_ref[...])
    f2e = jnp.maximum(_dot(emb_ref[...], sw_ref[...]) + sb_ref[...]
                      + agg_ref[...], 0.0)
    z = (_dot(emb_ref[...], w1_ref[...]) + q2e
         + _dot(f2e, w3_ref[...]) * FS + eb_ref[...])
    new_emb = jnp.maximum(z, 0.0)
    new_pr = PL * _col(agg_ref[...]) + (1.0 - PL) * pr_ref[...]
    hh = _dot(new_emb, hw_ref[...]) + hb_ref[...]
    ratio = new_pr / e2f_ref[...]
    colm = lax.broadcasted_iota(jnp.int32, (EP, DP), 1) == D
    emb_o[...] = new_emb
    pr_o[...] = new_pr
    hh_o[...] = jnp.where(colm, ratio, hh)


def _final_body(agg_ref, emb_ref, node_ref, le_ref, qw_ref, qb_ref, sw_ref,
                sb_ref, w1_ref, w2_ref, w3_ref, eb_ref, scw_ref, scb_ref,
                o_ref):
    nb = node_ref[...].reshape(1, DP)
    q2e = _dot(_dot(nb, qw_ref[...]) + qb_ref[...], w2_ref[...])
    f2e = jnp.maximum(_dot(emb_ref[...], sw_ref[...]) + sb_ref[...]
                      + agg_ref[...], 0.0)
    z = (_dot(emb_ref[...], w1_ref[...]) + q2e
         + _dot(f2e, w3_ref[...]) * FS + eb_ref[...])
    new_emb = jnp.maximum(z, 0.0)
    sc = lax.dot_general(scw_ref[...], new_emb,
                         (((1,), (1,)), ((), ())),
                         preferred_element_type=_f32) + scb_ref[...]
    mask = (le_ref[...].reshape(1, EP) != NE).astype(_f32)
    o_ref[...] = (sc + (1.0 - mask) * VERY_NEG)[None]


def _wspec(shape):
    nd = len(shape)
    return pl.BlockSpec(shape, lambda *a, _nd=nd: (0,) * _nd)


def _tc_proj(x, wt, b, bm):
    n, kk = x.shape
    return pl.pallas_call(
        _proj_body, grid=(pl.cdiv(n, bm),),
        in_specs=[pl.BlockSpec((bm, kk), lambda i: (i, 0)),
                  _wspec(wt.shape), _wspec(b.shape)],
        out_specs=pl.BlockSpec((bm, DP), lambda i: (i, 0)),
        out_shape=jax.ShapeDtypeStruct((n, DP), _f32))(x, wt, b)


def _tc_lstm(word_rows, wih, whh, b):
    return pl.pallas_call(
        _lstm_body,
        in_specs=[_wspec(word_rows.shape), _wspec(wih.shape),
                  _wspec(whh.shape), _wspec(b.shape)],
        out_specs=[_wspec((Q, 8, DP)), _wspec((8, DP))],
        out_shape=[jax.ShapeDtypeStruct((Q, 8, DP), _f32),
                   jax.ShapeDtypeStruct((8, DP), _f32)])(
                       word_rows, wih, whh, b)


def _tc_att(hs_t, fact_emb, qm3):
    return pl.pallas_call(
        _att_body, grid=(B,),
        in_specs=[pl.BlockSpec((1, Q, DP), lambda b: (b, 0, 0)),
                  pl.BlockSpec((FP, DP), lambda b: (b, 0)),
                  pl.BlockSpec((1, 1, Q), lambda b: (b, 0, 0))],
        out_specs=pl.BlockSpec((1, 1, FP), lambda b: (b, 0, 0)),
        out_shape=jax.ShapeDtypeStruct((B, 1, FP), _f32))(hs_t, fact_emb, qm3)


def _tc_e0(ent_emb, partials, pr0, hw, hb):
    return pl.pallas_call(
        _e0_body, grid=(B,),
        in_specs=[pl.BlockSpec((EP, DP), lambda b: (b, 0)),
                  pl.BlockSpec((NS, EP), lambda b: (b // 4, b % 4)),
                  pl.BlockSpec((EP, 1), lambda b: (b, 0)),
                  _wspec((DP, DP)), _wspec((1, DP))],
        out_specs=[pl.BlockSpec((EP, DP), lambda b: (b, 0)),
                   pl.BlockSpec((EP, 1), lambda b: (b, 0))],
        out_shape=[jax.ShapeDtypeStruct((EB, DP), _f32),
                   jax.ShapeDtypeStruct((EB, 1), _f32)])(
                       ent_emb, partials, pr0, hw, hb)


def _tc_payload(g, fact_emb, wt_col, sw, sb, tw, tb):
    bm = 4096
    return pl.pallas_call(
        _payload_body, grid=(FB // bm,),
        in_specs=[pl.BlockSpec((bm, DP), lambda i: (i, 0)),
                  pl.BlockSpec((bm, DP), lambda i: (i, 0)),
                  pl.BlockSpec((bm, 1), lambda i: (i, 0)),
                  _wspec((DP, DP)), _wspec((1, DP)),
                  _wspec((DP, DP)), _wspec((1, DP))],
        out_specs=pl.BlockSpec((bm, DP), lambda i: (i, 0)),
        out_shape=jax.ShapeDtypeStruct((FB, DP), _f32))(
            g, fact_emb, wt_col, sw, sb, tw, tb)


def _tc_combine(agg, ent_emb, pr, e2f, node, qw, qb, sw, sb, w1, w2, w3,
                eb, hw, hb):
    eblk = lambda b: (b, 0)
    return pl.pallas_call(
        _combine_body, grid=(B,),
        in_specs=[pl.BlockSpec((EP, DP), eblk), pl.BlockSpec((EP, DP), eblk),
                  pl.BlockSpec((EP, 1), eblk), pl.BlockSpec((EP, 1), eblk),
                  pl.BlockSpec((1, 1, DP), lambda b: (b, 0, 0)),
                  _wspec((DP, DP)), _wspec((1, DP)),
                  _wspec((DP, DP)), _wspec((1, DP)), _wspec((DP, DP)),
                  _wspec((DP, DP)), _wspec((DP, DP)), _wspec((1, DP)),
                  _wspec((DP, DP)), _wspec((1, DP))],
        out_specs=[pl.BlockSpec((EP, DP), eblk), pl.BlockSpec((EP, 1), eblk),
                   pl.BlockSpec((EP, DP), eblk)],
        out_shape=[jax.ShapeDtypeStruct((EB, DP), _f32),
                   jax.ShapeDtypeStruct((EB, 1), _f32),
                   jax.ShapeDtypeStruct((EB, DP), _f32)])(
                       agg, ent_emb, pr, e2f, node, qw, qb, sw, sb,
                       w1, w2, w3, eb, hw, hb)


def _tc_final(agg, ent_emb, node, le3, qw, qb, sw, sb, w1, w2, w3, eb,
              scw, scb):
    eblk = lambda b: (b, 0)
    return pl.pallas_call(
        _final_body, grid=(B,),
        in_specs=[pl.BlockSpec((EP, DP), eblk), pl.BlockSpec((EP, DP), eblk),
                  pl.BlockSpec((1, 1, DP), lambda b: (b, 0, 0)),
                  pl.BlockSpec((1, 1, EP), lambda b: (b, 0, 0)),
                  _wspec((DP, DP)), _wspec((1, DP)),
                  _wspec((DP, DP)), _wspec((1, DP)), _wspec((DP, DP)),
                  _wspec((DP, DP)), _wspec((DP, DP)), _wspec((1, DP)),
                  _wspec((1, DP)), _wspec((1, 1))],
        out_specs=pl.BlockSpec((1, 1, EP), lambda b: (b, 0, 0)),
        out_shape=jax.ShapeDtypeStruct((B, 1, EP), _f32))(
            agg, ent_emb, node, le3, qw, qb, sw, sb, w1, w2, w3, eb,
            scw, scb)


def _padT(w):
    o, i = w.shape
    return jnp.zeros((((i + 127) // 128) * 128, DP), _f32).at[:i, :o].set(w.T)


def _padTo(w):
    o, i = w.shape
    return jnp.zeros((i, DP), _f32).at[:, :o].set(w.T)


def _padb(b):
    return jnp.zeros((1, DP), _f32).at[0, :b.shape[0]].set(b)


def kernel(local_entity, q2e_adj_mat, kb_fact_rel, query_text, head_idx,
           tail_idx, entity_table, relation_table, word_table, ent_W, ent_b,
           rel_W, rel_b, lstm_Wih, lstm_Whh, lstm_b, q2e_W, q2e_b,
           kb_head_W, kb_head_b, kb_tail_W, kb_tail_b, kb_self_W, kb_self_b,
           e2e_W, e2e_b, score_W, score_b):
    local_entity = local_entity.astype(jnp.int32)
    kb_fact_rel = kb_fact_rel.astype(jnp.int32)
    query_text = query_text.astype(jnp.int32)
    head_idx = head_idx.astype(jnp.int32)
    tail_idx = tail_idx.astype(jnp.int32)

    entWt, entb = _padTo(ent_W), _padb(ent_b)
    relWt, relb = _padTo(rel_W), _padb(rel_b)
    wih = jnp.zeros((DP, 4, DP), _f32).at[:D, :, :D].set(
        lstm_Wih.reshape(4, D, D).transpose(2, 0, 1)).reshape(DP, 4 * DP)
    whh = jnp.zeros((DP, 4, DP), _f32).at[:D, :, :D].set(
        lstm_Whh.reshape(4, D, D).transpose(2, 0, 1)).reshape(DP, 4 * DP)
    lstmb = jnp.zeros((4, DP), _f32).at[:, :D].set(
        lstm_b.reshape(4, D)).reshape(1, 4 * DP)
    qWt = [_padT(q2e_W[i]) for i in range(NL)]
    qb = [_padb(q2e_b[i]) for i in range(NL)]
    hWt = [_padT(kb_head_W[i]) for i in range(NL)]
    hb = [_padb(kb_head_b[i]) for i in range(NL)]
    tWt = [_padT(kb_tail_W[i]) for i in range(NL)]
    tb = [_padb(kb_tail_b[i]) for i in range(NL)]
    sWt = [_padT(kb_self_W[i]) for i in range(NL)]
    sb = [_padb(kb_self_b[i]) for i in range(NL)]
    e1t = [_padT(e2e_W[i][:, 0 * D:1 * D]) for i in range(NL)]
    e2t = [_padT(e2e_W[i][:, 1 * D:2 * D]) for i in range(NL)]
    e3t = [_padT(e2e_W[i][:, 2 * D:3 * D]) for i in range(NL)]
    eb = [_padb(e2e_b[i]) for i in range(NL)]
    scw = jnp.zeros((1, DP), _f32).at[0, :D].set(score_W[0])
    scb = score_b.reshape(1, 1)

    le_pad = jnp.pad(local_entity, ((0, 0), (0, EP - E)), constant_values=NE)
    ent_gidx = le_pad.reshape(-1)
    word_idx = jnp.pad(query_text.T.reshape(-1), (0, 128),
                       constant_values=NW)
    rel_idx = jnp.pad(kb_fact_rel, ((0, 0), (0, FP - F)),
                      constant_values=NR).reshape(-1)
    head_pad = jnp.pad(head_idx, ((0, 0), (0, FP - F)))
    tail_pad = jnp.pad(tail_idx, ((0, 0), (0, FP - F)))
    boff = (jnp.arange(B, dtype=jnp.int32) * EP)[:, None]
    goff = ((jnp.arange(B, dtype=jnp.int32) % 4) * EP)[:, None]
    head_gidx = (head_pad + boff).reshape(-1)
    head_lidx = (head_pad + goff).reshape(-1)
    tail_lidx2d = (tail_pad + goff).reshape(-1).reshape(FB // 128, 128)
    pr0 = jnp.pad(q2e_adj_mat[:, :, 0], ((0, 0), (0, EP - E))).reshape(EB, 1)
    qmt = (query_text.T != NW).astype(_f32)
    le3 = le_pad.reshape(B, 1, EP)
    zeros512 = jnp.zeros((512, DP), _f32)

    ent_tab = _tc_proj(entity_table, entWt, entb, 4096)
    rel_tab = _tc_proj(relation_table, relWt, relb, 512)
    word_tab = _tc_pad(word_table, 8192)
    ent_emb = _sc_gather(ent_tab, ent_gidx, 512)
    word_rows = _sc_gather(word_tab, word_idx, 8)
    fact_emb = _sc_gather(rel_tab, rel_idx, 256)

    hs, node = _tc_lstm(word_rows, wih, whh, lstmb)
    hs_t = jnp.transpose(hs, (1, 0, 2))
    node3 = node.reshape(B, 1, DP)
    wt = _tc_att(hs_t, fact_emb, qmt.T.reshape(B, 1, Q))
    wt_flat = wt.reshape(-1)
    wt_col = wt.reshape(FB, 1)

    partials = _sc_scalar_scatter(head_lidx, wt_flat)
    hh, e2f = _tc_e0(ent_emb, partials, pr0, hWt[0], hb[0])
    pr = pr0

    for i in range(NL):
        g = _sc_gather(hh, head_gidx, 512)
        payload = _tc_payload(g, fact_emb, wt_col, sWt[i], sb[i],
                              tWt[i], tb[i])
        agg = _sc_scatter(zeros512, payload, tail_lidx2d)
        if i < NL - 1:
            ent_emb, pr, hh = _tc_combine(
                agg, ent_emb, pr, e2f, node3, qWt[i], qb[i], sWt[i], sb[i],
                e1t[i], e2t[i], e3t[i], eb[i], hWt[i + 1], hb[i + 1])
        else:
            score = _tc_final(
                agg, ent_emb, node3, le3, qWt[i], qb[i], sWt[i], sb[i],
                e1t[i], e2t[i], e3t[i], eb[i], scw, scb)

    return score.reshape(B, EP)[:, :E]


# speedup vs baseline: 7.9354x; 1.0434x over previous
import functools

import jax
import jax.numpy as jnp
from jax import lax
from jax.experimental import pallas as pl
from jax.experimental.pallas import tpu as pltpu
from jax.experimental.pallas import tpu_sc as plsc

B, E, F, Q, D = 8, 2000, 8000, 16, 100
EP, FP, DP = 2048, 8192, 128
EB, FB = B * EP, B * FP
NE, NR, NW = 200000, 500, 50000
NL = 3
NC, NS = 2, 16
VERY_NEG = -1e11
PL = 0.8
FS = 3.0
_f32 = jnp.float32


def _sc_gather(table, idx, CH):
    N = idx.shape[0]
    DW = table.shape[1]
    n = N // (NC * NS)
    nch = n // CH
    mesh = plsc.VectorSubcoreMesh(core_axis_name="c", subcore_axis_name="s")

    @functools.partial(
        pl.kernel, mesh=mesh,
        out_type=jax.ShapeDtypeStruct((N, DW), _f32),
        scratch_types=[pltpu.VMEM((n,), jnp.int32),
                       pltpu.VMEM((CH, DW), _f32),
                       pltpu.VMEM((CH, DW), _f32),
                       pltpu.SemaphoreType.DMA,
                       pltpu.SemaphoreType.DMA,
                       pltpu.SemaphoreType.DMA])
    def k(tab, ix, out, idx_v, rv0, rv1, g0, g1, so):
        wid = lax.axis_index("s") * NC + lax.axis_index("c")
        base = wid * n
        pltpu.sync_copy(ix.at[pl.ds(base, n)], idx_v)
        bufs = (rv0, rv1)
        gsems = (g0, g1)
        descs = [None] * nch
        descs[0] = pltpu.async_copy(tab.at[idx_v.at[pl.ds(0, CH)]],
                                    bufs[0], gsems[0])
        prev = None
        for j in range(nch):
            descs[j].wait()
            if prev is not None:
                prev.wait()
            if j + 1 < nch:
                descs[j + 1] = pltpu.async_copy(
                    tab.at[idx_v.at[pl.ds((j + 1) * CH, CH)]],
                    bufs[(j + 1) % 2], gsems[(j + 1) % 2])
            prev = pltpu.async_copy(bufs[j % 2],
                                    out.at[pl.ds(base + j * CH, CH)], so)
        prev.wait()

    return k(table, idx)


def _sc_scatter(zeros, msgs, idx2d):
    EG, CH = 4 * EP, 128
    NCH = 16
    mesh = plsc.VectorSubcoreMesh(core_axis_name="c", subcore_axis_name="s")

    @functools.partial(
        pl.kernel, mesh=mesh,
        out_type=jax.ShapeDtypeStruct((2 * EG, DP), _f32),
        scratch_types=[pltpu.VMEM((CH,), jnp.int32),
                       pltpu.VMEM((CH,), jnp.int32),
                       pltpu.VMEM((CH, DP), _f32),
                       pltpu.VMEM((CH, DP), _f32),
                       pltpu.VMEM_SHARED((EG, DP), _f32),
                       pltpu.SemaphoreType.DMA,
                       pltpu.SemaphoreType.DMA,
                       pltpu.SemaphoreType.DMA])
    def k(zer, ms, ix2, out, ix0, ix1, bf0, bf1, acc, l0, l1, ssem):
        c = lax.axis_index("c")
        s = lax.axis_index("s")
        pltpu.sync_copy(zer, acc.at[pl.ds(s * 512, 512)])
        plsc.subcore_barrier()
        base = c * 256 + s * 16
        ixs = (ix0, ix1)
        bfs = (bf0, bf1)
        lsems = (l0, l1)

        def load(j):
            chunk = base + j
            d1 = pltpu.async_copy(ms.at[pl.ds(chunk * CH, CH)],
                                  bfs[j % 2], lsems[j % 2])
            d2 = pltpu.async_copy(ix2.at[chunk], ixs[j % 2], lsems[j % 2])
            return (d1, d2)

        descs = [None] * NCH
        descs[0] = load(0)
        scat = [None] * NCH
        for j in range(NCH):
            descs[j][0].wait()
            descs[j][1].wait()
            if j >= 1:
                scat[j - 1].wait()
            if j + 1 < NCH:
                descs[j + 1] = load(j + 1)
            scat[j] = pltpu.async_copy(bfs[j % 2], acc.at[ixs[j % 2]],
                                       ssem, add=True)
        scat[NCH - 1].wait()
        plsc.subcore_barrier()
        pltpu.sync_copy(acc.at[pl.ds(s * 512, 512)],
                        out.at[pl.ds(c * EG + s * 512, 512)])

    return k(zeros, msgs, idx2d)


def _sc_scalar_scatter(idx, val):
    EG, n = 4 * EP, FB // (NC * NS)
    mesh = plsc.VectorSubcoreMesh(core_axis_name="c", subcore_axis_name="s")

    @functools.partial(
        pl.kernel, mesh=mesh,
        compiler_params=pltpu.CompilerParams(needs_layout_passes=False),
        out_type=jax.ShapeDtypeStruct((2 * NS, EG), _f32),
        scratch_types=[pltpu.VMEM((n,), jnp.int32),
                       pltpu.VMEM((n,), _f32),
                       pltpu.VMEM((EG,), _f32)])
    def k(ix, vl, out, idx_v, val_v, acc):
        c = lax.axis_index("c")
        s = lax.axis_index("s")
        base = (c * NS + s) * n

        def zbody(i, carry):
            acc[pl.ds(i * 16, 16)] = jnp.zeros((16,), _f32)
            return carry

        lax.fori_loop(0, EG // 16, zbody, 0)
        pltpu.sync_copy(ix.at[pl.ds(base, n)], idx_v)
        pltpu.sync_copy(vl.at[pl.ds(base, n)], val_v)

        def body(j, carry):
            iv = idx_v[pl.ds(j * 16, 16)]
            vv = jnp.maximum(val_v[pl.ds(j * 16, 16)], 0.0)
            plsc.addupdate_scatter(acc, [iv], vv)
            return carry

        lax.fori_loop(0, n // 16, body, 0)
        pltpu.sync_copy(acc, out.at[c * NS + s])

    return k(idx, val)


def _dot(a, b):
    return jnp.dot(a, b, preferred_element_type=_f32)


def _sig(x):
    return 1.0 / (1.0 + jnp.exp(-x))


def _proj_body(x_ref, w_ref, b_ref, o_ref):
    o_ref[...] = _dot(x_ref[...], w_ref[...]) + b_ref[...]


def _pad_body(x_ref, o_ref):
    o_ref[...] = jnp.pad(x_ref[...], ((0, 0), (0, DP - D)))


def _tc_pad(x, bm):
    n = x.shape[0]
    return pl.pallas_call(
        _pad_body, grid=(pl.cdiv(n, bm),),
        in_specs=[pl.BlockSpec((bm, D), lambda i: (i, 0))],
        out_specs=pl.BlockSpec((bm, DP), lambda i: (i, 0)),
        out_shape=jax.ShapeDtypeStruct((n, DP), _f32))(x)


def _lstm_body(wr_ref, wih_ref, whh_ref, b_ref, hs_ref, node_ref):
    def step(q, carry):
        h, c = carry
        x = wr_ref[pl.ds(q * 8, 8), :]
        z = _dot(x, wih_ref[...]) + _dot(h, whh_ref[...]) + b_ref[...]
        i = _sig(z[:, 0:DP])
        f = _sig(z[:, DP:2 * DP])
        g = jnp.tanh(z[:, 2 * DP:3 * DP])
        o = _sig(z[:, 3 * DP:4 * DP])
        c = f * c + i * g
        h = o * jnp.tanh(c)
        hs_ref[pl.ds(q, 1)] = h[None]
        return (h, c)

    z0 = jnp.zeros((8, DP), _f32)
    h, _ = lax.fori_loop(0, Q, step, (z0, z0))
    node_ref[...] = h


def _att_body(hs_ref, fact_ref, qmt_ref, o_ref):
    h = hs_ref[...].reshape(Q, DP)
    pen_row = qmt_ref[...].reshape(1, Q)
    pen = lax.dot_general(jnp.eye(Q, dtype=_f32), pen_row,
                          (((1,), (1,)), ((), ())),
                          preferred_element_type=_f32)
    sim = lax.dot_general(h, fact_ref[...],
                          (((1,), (1,)), ((), ())),
                          preferred_element_type=_f32) * (1.0 / 10.0)
    masked = sim + (1.0 - pen) * VERY_NEG
    mx = jnp.max(masked, axis=0, keepdims=True)
    ex = jnp.exp(masked - mx)
    p = ex / jnp.sum(ex, axis=0, keepdims=True)
    wf = jnp.sum(p * sim, axis=0, keepdims=True)
    fmask = lax.broadcasted_iota(jnp.int32, (1, FP), 1) < F
    wmax = jnp.max(jnp.where(fmask, wf, -1e30))
    wt = jnp.exp(wf - wmax)
    o_ref[...] = jnp.where(fmask, wt, -1.0)[None]


def _e0_body(emb_ref, part_ref, pr_ref, w_ref, b_ref, hh_ref, e2f_ref):
    ones = jnp.ones((Q, 1), _f32)
    psum = lax.dot_general(part_ref[...], ones,
                           (((0,), (0,)), ((), ())),
                           preferred_element_type=_f32)
    e2f = jnp.maximum(psum, 1e-10)
    ratio = pr_ref[...] / e2f
    hh = _dot(emb_ref[...], w_ref[...]) + b_ref[...]
    colm = lax.broadcasted_iota(jnp.int32, (EP, DP), 1) == D
    hh_ref[...] = jnp.where(colm, ratio, hh)
    e2f_ref[...] = e2f


def _col(x):
    m = lax.broadcasted_iota(jnp.int32, x.shape, 1) == D
    return jnp.sum(jnp.where(m, x, 0.0), axis=1, keepdims=True)


def _payload_body(g_ref, fact_ref, wt_ref, sw_ref, sb_ref, tw_ref, tb_ref,
                  o_ref):
    wt = wt_ref[...]
    valid = wt >= 0.0
    s = jnp.maximum(wt, 0.0) * _col(g_ref[...])
    e2f = jnp.maximum(_dot(fact_ref[...], sw_ref[...]) + sb_ref[...]
                      + g_ref[...], 0.0)
    th = _dot(e2f, tw_ref[...])
    pay = jnp.where(valid, th * s + tb_ref[...], 0.0)
    colm = lax.broadcasted_iota(jnp.int32, (pay.shape[0], DP), 1) == D
    o_ref[...] = jnp.where(colm, s, pay)


def _combine_body(agg_ref, emb_ref, pr_ref, e2f_ref, node_ref, qw_ref,
                  qb_ref, sw_ref, sb_ref, w1_ref, w2_ref, w3_ref, eb_ref,
                  hw_ref, hb_ref, emb_o, pr_o, hh_o):
    nb = node_ref[...].reshape(1, DP)
    q2e = _dot(_dot(nb, qw_ref[...]) + qb_ref[...], w2_ref[...])
    f2e = jnp.maximum(_dot(emb_ref[...], sw_ref[...]) + sb_ref[...]
                      + agg_ref[...], 0.0)
    z = (_dot(emb_ref[...], w1_ref[...]) + q2e
         + _dot(f2e, w3_ref[...]) * FS + eb_ref[...])
    new_emb = jnp.maximum(z, 0.0)
    new_pr = PL * _col(agg_ref[...]) + (1.0 - PL) * pr_ref[...]
    hh = _dot(new_emb, hw_ref[...]) + hb_ref[...]
    ratio = new_pr / e2f_ref[...]
    colm = lax.broadcasted_iota(jnp.int32, (EP, DP), 1) == D
    emb_o[...] = new_emb
    pr_o[...] = new_pr
    hh_o[...] = jnp.where(colm, ratio, hh)


def _final_body(agg_ref, emb_ref, node_ref, le_ref, qw_ref, qb_ref, sw_ref,
                sb_ref, w1_ref, w2_ref, w3_ref, eb_ref, scw_ref, scb_ref,
                o_ref):
    nb = node_ref[...].reshape(1, DP)
    q2e = _dot(_dot(nb, qw_ref[...]) + qb_ref[...], w2_ref[...])
    f2e = jnp.maximum(_dot(emb_ref[...], sw_ref[...]) + sb_ref[...]
                      + agg_ref[...], 0.0)
    z = (_dot(emb_ref[...], w1_ref[...]) + q2e
         + _dot(f2e, w3_ref[...]) * FS + eb_ref[...])
    new_emb = jnp.maximum(z, 0.0)
    sc = lax.dot_general(scw_ref[...], new_emb,
                         (((1,), (1,)), ((), ())),
                         preferred_element_type=_f32) + scb_ref[...]
    mask = (le_ref[...].reshape(1, EP) != NE).astype(_f32)
    o_ref[...] = (sc + (1.0 - mask) * VERY_NEG)[None]


def _wspec(shape):
    nd = len(shape)
    return pl.BlockSpec(shape, lambda *a, _nd=nd: (0,) * _nd)


def _tc_proj(x, wt, b, bm):
    n, kk = x.shape
    return pl.pallas_call(
        _proj_body, grid=(pl.cdiv(n, bm),),
        in_specs=[pl.BlockSpec((bm, kk), lambda i: (i, 0)),
                  _wspec(wt.shape), _wspec(b.shape)],
        out_specs=pl.BlockSpec((bm, DP), lambda i: (i, 0)),
        out_shape=jax.ShapeDtypeStruct((n, DP), _f32))(x, wt, b)


def _tc_lstm(word_rows, wih, whh, b):
    return pl.pallas_call(
        _lstm_body,
        in_specs=[_wspec(word_rows.shape), _wspec(wih.shape),
                  _wspec(whh.shape), _wspec(b.shape)],
        out_specs=[_wspec((Q, 8, DP)), _wspec((8, DP))],
        out_shape=[jax.ShapeDtypeStruct((Q, 8, DP), _f32),
                   jax.ShapeDtypeStruct((8, DP), _f32)])(
                       word_rows, wih, whh, b)


def _tc_att(hs_t, fact_emb, qm3):
    return pl.pallas_call(
        _att_body, grid=(B,),
        in_specs=[pl.BlockSpec((1, Q, DP), lambda b: (b, 0, 0)),
                  pl.BlockSpec((FP, DP), lambda b: (b, 0)),
                  pl.BlockSpec((1, 1, Q), lambda b: (b, 0, 0))],
        out_specs=pl.BlockSpec((1, 1, FP), lambda b: (b, 0, 0)),
        out_shape=jax.ShapeDtypeStruct((B, 1, FP), _f32))(hs_t, fact_emb, qm3)


def _tc_e0(ent_emb, partials, pr0, hw, hb):
    return pl.pallas_call(
        _e0_body, grid=(B,),
        in_specs=[pl.BlockSpec((EP, DP), lambda b: (b, 0)),
                  pl.BlockSpec((NS, EP), lambda b: (b // 4, b % 4)),
                  pl.BlockSpec((EP, 1), lambda b: (b, 0)),
                  _wspec((DP, DP)), _wspec((1, DP))],
        out_specs=[pl.BlockSpec((EP, DP), lambda b: (b, 0)),
                   pl.BlockSpec((EP, 1), lambda b: (b, 0))],
        out_shape=[jax.ShapeDtypeStruct((EB, DP), _f32),
                   jax.ShapeDtypeStruct((EB, 1), _f32)])(
                       ent_emb, partials, pr0, hw, hb)


def _tc_payload(g, fact_emb, wt_col, sw, sb, tw, tb):
    bm = 4096
    return pl.pallas_call(
        _payload_body, grid=(FB // bm,),
        in_specs=[pl.BlockSpec((bm, DP), lambda i: (i, 0)),
                  pl.BlockSpec((bm, DP), lambda i: (i, 0)),
                  pl.BlockSpec((bm, 1), lambda i: (i, 0)),
                  _wspec((DP, DP)), _wspec((1, DP)),
                  _wspec((DP, DP)), _wspec((1, DP))],
        out_specs=pl.BlockSpec((bm, DP), lambda i: (i, 0)),
        out_shape=jax.ShapeDtypeStruct((FB, DP), _f32))(
            g, fact_emb, wt_col, sw, sb, tw, tb)


def _tc_combine(agg, ent_emb, pr, e2f, node, qw, qb, sw, sb, w1, w2, w3,
                eb, hw, hb):
    eblk = lambda b: (b, 0)
    return pl.pallas_call(
        _combine_body, grid=(B,),
        in_specs=[pl.BlockSpec((EP, DP), eblk), pl.BlockSpec((EP, DP), eblk),
                  pl.BlockSpec((EP, 1), eblk), pl.BlockSpec((EP, 1), eblk),
                  pl.BlockSpec((1, 1, DP), lambda b: (b, 0, 0)),
                  _wspec((DP, DP)), _wspec((1, DP)),
                  _wspec((DP, DP)), _wspec((1, DP)), _wspec((DP, DP)),
                  _wspec((DP, DP)), _wspec((DP, DP)), _wspec((1, DP)),
                  _wspec((DP, DP)), _wspec((1, DP))],
        out_specs=[pl.BlockSpec((EP, DP), eblk), pl.BlockSpec((EP, 1), eblk),
                   pl.BlockSpec((EP, DP), eblk)],
        out_shape=[jax.ShapeDtypeStruct((EB, DP), _f32),
                   jax.ShapeDtypeStruct((EB, 1), _f32),
                   jax.ShapeDtypeStruct((EB, DP), _f32)])(
                       agg, ent_emb, pr, e2f, node, qw, qb, sw, sb,
                       w1, w2, w3, eb, hw, hb)


def _tc_final(agg, ent_emb, node, le3, qw, qb, sw, sb, w1, w2, w3, eb,
              scw, scb):
    eblk = lambda b: (b, 0)
    return pl.pallas_call(
        _final_body, grid=(B,),
        in_specs=[pl.BlockSpec((EP, DP), eblk), pl.BlockSpec((EP, DP), eblk),
                  pl.BlockSpec((1, 1, DP), lambda b: (b, 0, 0)),
                  pl.BlockSpec((1, 1, EP), lambda b: (b, 0, 0)),
                  _wspec((DP, DP)), _wspec((1, DP)),
                  _wspec((DP, DP)), _wspec((1, DP)), _wspec((DP, DP)),
                  _wspec((DP, DP)), _wspec((DP, DP)), _wspec((1, DP)),
                  _wspec((1, DP)), _wspec((1, 1))],
        out_specs=pl.BlockSpec((1, 1, EP), lambda b: (b, 0, 0)),
        out_shape=jax.ShapeDtypeStruct((B, 1, EP), _f32))(
            agg, ent_emb, node, le3, qw, qb, sw, sb, w1, w2, w3, eb,
            scw, scb)


def _padT(w):
    o, i = w.shape
    return jnp.zeros((((i + 127) // 128) * 128, DP), _f32).at[:i, :o].set(w.T)


def _padTo(w):
    o, i = w.shape
    return jnp.zeros((i, DP), _f32).at[:, :o].set(w.T)


def _padb(b):
    return jnp.zeros((1, DP), _f32).at[0, :b.shape[0]].set(b)


def kernel(local_entity, q2e_adj_mat, kb_fact_rel, query_text, head_idx,
           tail_idx, entity_table, relation_table, word_table, ent_W, ent_b,
           rel_W, rel_b, lstm_Wih, lstm_Whh, lstm_b, q2e_W, q2e_b,
           kb_head_W, kb_head_b, kb_tail_W, kb_tail_b, kb_self_W, kb_self_b,
           e2e_W, e2e_b, score_W, score_b):
    local_entity = local_entity.astype(jnp.int32)
    kb_fact_rel = kb_fact_rel.astype(jnp.int32)
    query_text = query_text.astype(jnp.int32)
    head_idx = head_idx.astype(jnp.int32)
    tail_idx = tail_idx.astype(jnp.int32)

    entWt, entb = _padTo(ent_W), _padb(ent_b)
    relWt, relb = _padTo(rel_W), _padb(rel_b)
    wih = jnp.zeros((DP, 4, DP), _f32).at[:D, :, :D].set(
        lstm_Wih.reshape(4, D, D).transpose(2, 0, 1)).reshape(DP, 4 * DP)
    whh = jnp.zeros((DP, 4, DP), _f32).at[:D, :, :D].set(
        lstm_Whh.reshape(4, D, D).transpose(2, 0, 1)).reshape(DP, 4 * DP)
    lstmb = jnp.zeros((4, DP), _f32).at[:, :D].set(
        lstm_b.reshape(4, D)).reshape(1, 4 * DP)
    qWt = [_padT(q2e_W[i]) for i in range(NL)]
    qb = [_padb(q2e_b[i]) for i in range(NL)]
    hWt = [_padT(kb_head_W[i]) for i in range(NL)]
    hb = [_padb(kb_head_b[i]) for i in range(NL)]
    tWt = [_padT(kb_tail_W[i]) for i in range(NL)]
    tb = [_padb(kb_tail_b[i]) for i in range(NL)]
    sWt = [_padT(kb_self_W[i]) for i in range(NL)]
    sb = [_padb(kb_self_b[i]) for i in range(NL)]
    e1t = [_padT(e2e_W[i][:, 0 * D:1 * D]) for i in range(NL)]
    e2t = [_padT(e2e_W[i][:, 1 * D:2 * D]) for i in range(NL)]
    e3t = [_padT(e2e_W[i][:, 2 * D:3 * D]) for i in range(NL)]
    eb = [_padb(e2e_b[i]) for i in range(NL)]
    scw = jnp.zeros((1, DP), _f32).at[0, :D].set(score_W[0])
    scb = score_b.reshape(1, 1)

    le_pad = jnp.pad(local_entity, ((0, 0), (0, EP - E)), constant_values=NE)
    ent_gidx = le_pad.reshape(-1)
    word_idx = jnp.pad(query_text.T.reshape(-1), (0, 128),
                       constant_values=NW)
    rel_idx = jnp.pad(kb_fact_rel, ((0, 0), (0, FP - F)),
                      constant_values=NR).reshape(-1)
    head_pad = jnp.pad(head_idx, ((0, 0), (0, FP - F)))
    tail_pad = jnp.pad(tail_idx, ((0, 0), (0, FP - F)))
    boff = (jnp.arange(B, dtype=jnp.int32) * EP)[:, None]
    goff = ((jnp.arange(B, dtype=jnp.int32) % 4) * EP)[:, None]
    head_gidx = (head_pad + boff).reshape(-1)
    head_lidx = (head_pad + goff).reshape(-1)
    tail_lidx2d = (tail_pad + goff).reshape(-1).reshape(FB // 128, 128)
    pr0 = jnp.pad(q2e_adj_mat[:, :, 0], ((0, 0), (0, EP - E))).reshape(EB, 1)
    qmt = (query_text.T != NW).astype(_f32)
    le3 = le_pad.reshape(B, 1, EP)
    zeros512 = jnp.zeros((512, DP), _f32)

    ent_tab = _tc_proj(entity_table, entWt, entb, 4096)
    rel_tab = _tc_proj(relation_table, relWt, relb, 512)
    word_tab = _tc_pad(word_table, 8192)
    ent_emb = _sc_gather(ent_tab, ent_gidx, 128)
    word_rows = _sc_gather(word_tab, word_idx, 8)
    fact_emb = _sc_gather(rel_tab, rel_idx, 256)

    hs, node = _tc_lstm(word_rows, wih, whh, lstmb)
    hs_t = jnp.transpose(hs, (1, 0, 2))
    node3 = node.reshape(B, 1, DP)
    wt = _tc_att(hs_t, fact_emb, qmt.T.reshape(B, 1, Q))
    wt_flat = wt.reshape(-1)
    wt_col = wt.reshape(FB, 1)

    partials = _sc_scalar_scatter(head_lidx, wt_flat)
    hh, e2f = _tc_e0(ent_emb, partials, pr0, hWt[0], hb[0])
    pr = pr0

    for i in range(NL):
        g = _sc_gather(hh, head_gidx, 256)
        payload = _tc_payload(g, fact_emb, wt_col, sWt[i], sb[i],
                              tWt[i], tb[i])
        agg = _sc_scatter(zeros512, payload, tail_lidx2d)
        if i < NL - 1:
            ent_emb, pr, hh = _tc_combine(
                agg, ent_emb, pr, e2f, node3, qWt[i], qb[i], sWt[i], sb[i],
                e1t[i], e2t[i], e3t[i], eb[i], hWt[i + 1], hb[i + 1])
        else:
            score = _tc_final(
                agg, ent_emb, node3, le3, qWt[i], qb[i], sWt[i], sb[i],
                e1t[i], e2t[i], e3t[i], eb[i], scw, scb)

    return score.reshape(B, EP)[:, :E]


# speedup vs baseline: 8.0345x; 1.0125x over previous
import functools

import jax
import jax.numpy as jnp
from jax import lax
from jax.experimental import pallas as pl
from jax.experimental.pallas import tpu as pltpu
from jax.experimental.pallas import tpu_sc as plsc

B, E, F, Q, D = 8, 2000, 8000, 16, 100
EP, FP, DP = 2048, 8192, 128
EB, FB = B * EP, B * FP
NE, NR, NW = 200000, 500, 50000
NL = 3
NC, NS = 2, 16
VERY_NEG = -1e11
PL = 0.8
FS = 3.0
_f32 = jnp.float32


def _sc_gather(table, idx, CH):
    N = idx.shape[0]
    DW = table.shape[1]
    n = N // (NC * NS)
    nch = n // CH
    NB = min(nch, 4)
    mesh = plsc.VectorSubcoreMesh(core_axis_name="c", subcore_axis_name="s")

    @functools.partial(
        pl.kernel, mesh=mesh,
        out_type=jax.ShapeDtypeStruct((N, DW), _f32),
        scratch_types=[pltpu.VMEM((n,), jnp.int32)]
        + [pltpu.VMEM((CH, DW), _f32)] * NB
        + [pltpu.SemaphoreType.DMA] * (2 * NB))
    def k(tab, ix, out, idx_v, *rest):
        bufs = rest[:NB]
        gsems = rest[NB:2 * NB]
        osems = rest[2 * NB:]
        wid = lax.axis_index("s") * NC + lax.axis_index("c")
        base = wid * n
        pltpu.sync_copy(ix.at[pl.ds(base, n)], idx_v)

        def gath(j):
            return pltpu.async_copy(
                tab.at[idx_v.at[pl.ds(j * CH, CH)]], bufs[j % NB],
                gsems[j % NB])

        descs = [None] * nch
        couts = [None] * nch
        for j in range(NB):
            descs[j] = gath(j)
        for j in range(nch):
            descs[j].wait()
            couts[j] = pltpu.async_copy(
                bufs[j % NB], out.at[pl.ds(base + j * CH, CH)],
                osems[j % NB])
            if j + NB < nch:
                couts[j].wait()
                descs[j + NB] = gath(j + NB)
        for j in range(max(0, nch - NB), nch):
            if couts[j] is not None and j + NB >= nch:
                couts[j].wait()

    return k(table, idx)


def _sc_scatter(zeros, msgs, idx2d):
    EG, CH = 4 * EP, 128
    NCH = 16
    mesh = plsc.VectorSubcoreMesh(core_axis_name="c", subcore_axis_name="s")

    @functools.partial(
        pl.kernel, mesh=mesh,
        out_type=jax.ShapeDtypeStruct((2 * EG, DP), _f32),
        scratch_types=[pltpu.VMEM((CH,), jnp.int32),
                       pltpu.VMEM((CH,), jnp.int32),
                       pltpu.VMEM((CH, DP), _f32),
                       pltpu.VMEM((CH, DP), _f32),
                       pltpu.VMEM_SHARED((EG, DP), _f32),
                       pltpu.SemaphoreType.DMA,
                       pltpu.SemaphoreType.DMA,
                       pltpu.SemaphoreType.DMA])
    def k(zer, ms, ix2, out, ix0, ix1, bf0, bf1, acc, l0, l1, ssem):
        c = lax.axis_index("c")
        s = lax.axis_index("s")
        pltpu.sync_copy(zer, acc.at[pl.ds(s * 512, 512)])
        plsc.subcore_barrier()
        base = c * 256 + s * 16
        ixs = (ix0, ix1)
        bfs = (bf0, bf1)
        lsems = (l0, l1)

        def load(j):
            chunk = base + j
            d1 = pltpu.async_copy(ms.at[pl.ds(chunk * CH, CH)],
                                  bfs[j % 2], lsems[j % 2])
            d2 = pltpu.async_copy(ix2.at[chunk], ixs[j % 2], lsems[j % 2])
            return (d1, d2)

        descs = [None] * NCH
        descs[0] = load(0)
        scat = [None] * NCH
        for j in range(NCH):
            descs[j][0].wait()
            descs[j][1].wait()
            if j >= 1:
                scat[j - 1].wait()
            if j + 1 < NCH:
                descs[j + 1] = load(j + 1)
            scat[j] = pltpu.async_copy(bfs[j % 2], acc.at[ixs[j % 2]],
                                       ssem, add=True)
        scat[NCH - 1].wait()
        plsc.subcore_barrier()
        pltpu.sync_copy(acc.at[pl.ds(s * 512, 512)],
                        out.at[pl.ds(c * EG + s * 512, 512)])

    return k(zeros, msgs, idx2d)


def _sc_scalar_scatter(idx, val):
    EG, n = 4 * EP, FB // (NC * NS)
    mesh = plsc.VectorSubcoreMesh(core_axis_name="c", subcore_axis_name="s")

    @functools.partial(
        pl.kernel, mesh=mesh,
        compiler_params=pltpu.CompilerParams(needs_layout_passes=False),
        out_type=jax.ShapeDtypeStruct((2 * NS, EG), _f32),
        scratch_types=[pltpu.VMEM((n,), jnp.int32),
                       pltpu.VMEM((n,), _f32),
                       pltpu.VMEM((EG,), _f32)])
    def k(ix, vl, out, idx_v, val_v, acc):
        c = lax.axis_index("c")
        s = lax.axis_index("s")
        base = (c * NS + s) * n

        def zbody(i, carry):
            acc[pl.ds(i * 16, 16)] = jnp.zeros((16,), _f32)
            return carry

        lax.fori_loop(0, EG // 16, zbody, 0)
        pltpu.sync_copy(ix.at[pl.ds(base, n)], idx_v)
        pltpu.sync_copy(vl.at[pl.ds(base, n)], val_v)

        def body(j, carry):
            iv = idx_v[pl.ds(j * 16, 16)]
            vv = jnp.maximum(val_v[pl.ds(j * 16, 16)], 0.0)
            plsc.addupdate_scatter(acc, [iv], vv)
            return carry

        lax.fori_loop(0, n // 16, body, 0)
        pltpu.sync_copy(acc, out.at[c * NS + s])

    return k(idx, val)


def _dot(a, b):
    return jnp.dot(a, b, preferred_element_type=_f32)


def _sig(x):
    return 1.0 / (1.0 + jnp.exp(-x))


def _proj_body(x_ref, w_ref, b_ref, o_ref):
    o_ref[...] = _dot(x_ref[...], w_ref[...]) + b_ref[...]


def _pad_body(x_ref, o_ref):
    o_ref[...] = jnp.pad(x_ref[...], ((0, 0), (0, DP - D)))


def _tc_pad(x, bm):
    n = x.shape[0]
    return pl.pallas_call(
        _pad_body, grid=(pl.cdiv(n, bm),),
        in_specs=[pl.BlockSpec((bm, D), lambda i: (i, 0))],
        out_specs=pl.BlockSpec((bm, DP), lambda i: (i, 0)),
        out_shape=jax.ShapeDtypeStruct((n, DP), _f32))(x)


def _lstm_body(wr_ref, wih_ref, whh_ref, b_ref, hs_ref, node_ref):
    def step(q, carry):
        h, c = carry
        x = wr_ref[pl.ds(q * 8, 8), :]
        z = _dot(x, wih_ref[...]) + _dot(h, whh_ref[...]) + b_ref[...]
        i = _sig(z[:, 0:DP])
        f = _sig(z[:, DP:2 * DP])
        g = jnp.tanh(z[:, 2 * DP:3 * DP])
        o = _sig(z[:, 3 * DP:4 * DP])
        c = f * c + i * g
        h = o * jnp.tanh(c)
        hs_ref[pl.ds(q, 1)] = h[None]
        return (h, c)

    z0 = jnp.zeros((8, DP), _f32)
    h, _ = lax.fori_loop(0, Q, step, (z0, z0))
    node_ref[...] = h


def _att_body(hs_ref, fact_ref, qmt_ref, o_ref):
    h = hs_ref[...].reshape(Q, DP)
    pen_row = qmt_ref[...].reshape(1, Q)
    pen = lax.dot_general(jnp.eye(Q, dtype=_f32), pen_row,
                          (((1,), (1,)), ((), ())),
                          preferred_element_type=_f32)
    sim = lax.dot_general(h, fact_ref[...],
                          (((1,), (1,)), ((), ())),
                          preferred_element_type=_f32) * (1.0 / 10.0)
    masked = sim + (1.0 - pen) * VERY_NEG
    mx = jnp.max(masked, axis=0, keepdims=True)
    ex = jnp.exp(masked - mx)
    p = ex / jnp.sum(ex, axis=0, keepdims=True)
    wf = jnp.sum(p * sim, axis=0, keepdims=True)
    fmask = lax.broadcasted_iota(jnp.int32, (1, FP), 1) < F
    wmax = jnp.max(jnp.where(fmask, wf, -1e30))
    wt = jnp.exp(wf - wmax)
    o_ref[...] = jnp.where(fmask, wt, -1.0)[None]


def _e0_body(emb_ref, part_ref, pr_ref, w_ref, b_ref, hh_ref, e2f_ref):
    ones = jnp.ones((Q, 1), _f32)
    psum = lax.dot_general(part_ref[...], ones,
                           (((0,), (0,)), ((), ())),
                           preferred_element_type=_f32)
    e2f = jnp.maximum(psum, 1e-10)
    ratio = pr_ref[...] / e2f
    hh = _dot(emb_ref[...], w_ref[...]) + b_ref[...]
    colm = lax.broadcasted_iota(jnp.int32, (EP, DP), 1) == D
    hh_ref[...] = jnp.where(colm, ratio, hh)
    e2f_ref[...] = e2f


def _col(x):
    m = lax.broadcasted_iota(jnp.int32, x.shape, 1) == D
    return jnp.sum(jnp.where(m, x, 0.0), axis=1, keepdims=True)


def _payload_body(g_ref, fact_ref, wt_ref, sw_ref, sb_ref, tw_ref, tb_ref,
                  o_ref):
    wt = wt_ref[...]
    valid = wt >= 0.0
    s = jnp.maximum(wt, 0.0) * _col(g_ref[...])
    e2f = jnp.maximum(_dot(fact_ref[...], sw_ref[...]) + sb_ref[...]
                      + g_ref[...], 0.0)
    th = _dot(e2f, tw_ref[...])
    pay = jnp.where(valid, th * s + tb_ref[...], 0.0)
    colm = lax.broadcasted_iota(jnp.int32, (pay.shape[0], DP), 1) == D
    o_ref[...] = jnp.where(colm, s, pay)


def _combine_body(agg_ref, emb_ref, pr_ref, e2f_ref, node_ref, qw_ref,
                  qb_ref, sw_ref, sb_ref, w1_ref, w2_ref, w3_ref, eb_ref,
                  hw_ref, hb_ref, emb_o, pr_o, hh_o):
    nb = node_ref[...].reshape(1, DP)
    q2e = _dot(_dot(nb, qw_ref[...]) + qb_ref[...], w2_ref[...])
    f2e = jnp.maximum(_dot(emb_ref[...], sw_ref[...]) + sb_ref[...]
                      + agg_ref[...], 0.0)
    z = (_dot(emb_ref[...], w1_ref[...]) + q2e
         + _dot(f2e, w3_ref[...]) * FS + eb_ref[...])
    new_emb = jnp.maximum(z, 0.0)
    new_pr = PL * _col(agg_ref[...]) + (1.0 - PL) * pr_ref[...]
    hh = _dot(new_emb, hw_ref[...]) + hb_ref[...]
    ratio = new_pr / e2f_ref[...]
    colm = lax.broadcasted_iota(jnp.int32, (EP, DP), 1) == D
    emb_o[...] = new_emb
    pr_o[...] = new_pr
    hh_o[...] = jnp.where(colm, ratio, hh)


def _final_body(agg_ref, emb_ref, node_ref, le_ref, qw_ref, qb_ref, sw_ref,
                sb_ref, w1_ref, w2_ref, w3_ref, eb_ref, scw_ref, scb_ref,
                o_ref):
    nb = node_ref[...].reshape(1, DP)
    q2e = _dot(_dot(nb, qw_ref[...]) + qb_ref[...], w2_ref[...])
    f2e = jnp.maximum(_dot(emb_ref[...], sw_ref[...]) + sb_ref[...]
                      + agg_ref[...], 0.0)
    z = (_dot(emb_ref[...], w1_ref[...]) + q2e
         + _dot(f2e, w3_ref[...]) * FS + eb_ref[...])
    new_emb = jnp.maximum(z, 0.0)
    sc = lax.dot_general(scw_ref[...], new_emb,
                         (((1,), (1,)), ((), ())),
                         preferred_element_type=_f32) + scb_ref[...]
    mask = (le_ref[...].reshape(1, EP) != NE).astype(_f32)
    o_ref[...] = (sc + (1.0 - mask) * VERY_NEG)[None]


def _wspec(shape):
    nd = len(shape)
    return pl.BlockSpec(shape, lambda *a, _nd=nd: (0,) * _nd)


def _tc_proj(x, wt, b, bm):
    n, kk = x.shape
    return pl.pallas_call(
        _proj_body, grid=(pl.cdiv(n, bm),),
        in_specs=[pl.BlockSpec((bm, kk), lambda i: (i, 0)),
                  _wspec(wt.shape), _wspec(b.shape)],
        out_specs=pl.BlockSpec((bm, DP), lambda i: (i, 0)),
        out_shape=jax.ShapeDtypeStruct((n, DP), _f32))(x, wt, b)


def _tc_lstm(word_rows, wih, whh, b):
    return pl.pallas_call(
        _lstm_body,
        in_specs=[_wspec(word_rows.shape), _wspec(wih.shape),
                  _wspec(whh.shape), _wspec(b.shape)],
        out_specs=[_wspec((Q, 8, DP)), _wspec((8, DP))],
        out_shape=[jax.ShapeDtypeStruct((Q, 8, DP), _f32),
                   jax.ShapeDtypeStruct((8, DP), _f32)])(
                       word_rows, wih, whh, b)


def _tc_att(hs_t, fact_emb, qm3):
    return pl.pallas_call(
        _att_body, grid=(B,),
        in_specs=[pl.BlockSpec((1, Q, DP), lambda b: (b, 0, 0)),
                  pl.BlockSpec((FP, DP), lambda b: (b, 0)),
                  pl.BlockSpec((1, 1, Q), lambda b: (b, 0, 0))],
        out_specs=pl.BlockSpec((1, 1, FP), lambda b: (b, 0, 0)),
        out_shape=jax.ShapeDtypeStruct((B, 1, FP), _f32))(hs_t, fact_emb, qm3)


def _tc_e0(ent_emb, partials, pr0, hw, hb):
    return pl.pallas_call(
        _e0_body, grid=(B,),
        in_specs=[pl.BlockSpec((EP, DP), lambda b: (b, 0)),
                  pl.BlockSpec((NS, EP), lambda b: (b // 4, b % 4)),
                  pl.BlockSpec((EP, 1), lambda b: (b, 0)),
                  _wspec((DP, DP)), _wspec((1, DP))],
        out_specs=[pl.BlockSpec((EP, DP), lambda b: (b, 0)),
                   pl.BlockSpec((EP, 1), lambda b: (b, 0))],
        out_shape=[jax.ShapeDtypeStruct((EB, DP), _f32),
                   jax.ShapeDtypeStruct((EB, 1), _f32)])(
                       ent_emb, partials, pr0, hw, hb)


def _tc_payload(g, fact_emb, wt_col, sw, sb, tw, tb):
    bm = 4096
    return pl.pallas_call(
        _payload_body, grid=(FB // bm,),
        in_specs=[pl.BlockSpec((bm, DP), lambda i: (i, 0)),
                  pl.BlockSpec((bm, DP), lambda i: (i, 0)),
                  pl.BlockSpec((bm, 1), lambda i: (i, 0)),
                  _wspec((DP, DP)), _wspec((1, DP)),
                  _wspec((DP, DP)), _wspec((1, DP))],
        out_specs=pl.BlockSpec((bm, DP), lambda i: (i, 0)),
        out_shape=jax.ShapeDtypeStruct((FB, DP), _f32))(
            g, fact_emb, wt_col, sw, sb, tw, tb)


def _tc_combine(agg, ent_emb, pr, e2f, node, qw, qb, sw, sb, w1, w2, w3,
                eb, hw, hb):
    eblk = lambda b: (b, 0)
    return pl.pallas_call(
        _combine_body, grid=(B,),
        in_specs=[pl.BlockSpec((EP, DP), eblk), pl.BlockSpec((EP, DP), eblk),
                  pl.BlockSpec((EP, 1), eblk), pl.BlockSpec((EP, 1), eblk),
                  pl.BlockSpec((1, 1, DP), lambda b: (b, 0, 0)),
                  _wspec((DP, DP)), _wspec((1, DP)),
                  _wspec((DP, DP)), _wspec((1, DP)), _wspec((DP, DP)),
                  _wspec((DP, DP)), _wspec((DP, DP)), _wspec((1, DP)),
                  _wspec((DP, DP)), _wspec((1, DP))],
        out_specs=[pl.BlockSpec((EP, DP), eblk), pl.BlockSpec((EP, 1), eblk),
                   pl.BlockSpec((EP, DP), eblk)],
        out_shape=[jax.ShapeDtypeStruct((EB, DP), _f32),
                   jax.ShapeDtypeStruct((EB, 1), _f32),
                   jax.ShapeDtypeStruct((EB, DP), _f32)])(
                       agg, ent_emb, pr, e2f, node, qw, qb, sw, sb,
                       w1, w2, w3, eb, hw, hb)


def _tc_final(agg, ent_emb, node, le3, qw, qb, sw, sb, w1, w2, w3, eb,
              scw, scb):
    eblk = lambda b: (b, 0)
    return pl.pallas_call(
        _final_body, grid=(B,),
        in_specs=[pl.BlockSpec((EP, DP), eblk), pl.BlockSpec((EP, DP), eblk),
                  pl.BlockSpec((1, 1, DP), lambda b: (b, 0, 0)),
                  pl.BlockSpec((1, 1, EP), lambda b: (b, 0, 0)),
                  _wspec((DP, DP)), _wspec((1, DP)),
                  _wspec((DP, DP)), _wspec((1, DP)), _wspec((DP, DP)),
                  _wspec((DP, DP)), _wspec((DP, DP)), _wspec((1, DP)),
                  _wspec((1, DP)), _wspec((1, 1))],
        out_specs=pl.BlockSpec((1, 1, EP), lambda b: (b, 0, 0)),
        out_shape=jax.ShapeDtypeStruct((B, 1, EP), _f32))(
            agg, ent_emb, node, le3, qw, qb, sw, sb, w1, w2, w3, eb,
            scw, scb)


def _padT(w):
    o, i = w.shape
    return jnp.zeros((((i + 127) // 128) * 128, DP), _f32).at[:i, :o].set(w.T)


def _padTo(w):
    o, i = w.shape
    return jnp.zeros((i, DP), _f32).at[:, :o].set(w.T)


def _padb(b):
    return jnp.zeros((1, DP), _f32).at[0, :b.shape[0]].set(b)


def kernel(local_entity, q2e_adj_mat, kb_fact_rel, query_text, head_idx,
           tail_idx, entity_table, relation_table, word_table, ent_W, ent_b,
           rel_W, rel_b, lstm_Wih, lstm_Whh, lstm_b, q2e_W, q2e_b,
           kb_head_W, kb_head_b, kb_tail_W, kb_tail_b, kb_self_W, kb_self_b,
           e2e_W, e2e_b, score_W, score_b):
    local_entity = local_entity.astype(jnp.int32)
    kb_fact_rel = kb_fact_rel.astype(jnp.int32)
    query_text = query_text.astype(jnp.int32)
    head_idx = head_idx.astype(jnp.int32)
    tail_idx = tail_idx.astype(jnp.int32)

    entWt, entb = _padTo(ent_W), _padb(ent_b)
    relWt, relb = _padTo(rel_W), _padb(rel_b)
    wih = jnp.zeros((DP, 4, DP), _f32).at[:D, :, :D].set(
        lstm_Wih.reshape(4, D, D).transpose(2, 0, 1)).reshape(DP, 4 * DP)
    whh = jnp.zeros((DP, 4, DP), _f32).at[:D, :, :D].set(
        lstm_Whh.reshape(4, D, D).transpose(2, 0, 1)).reshape(DP, 4 * DP)
    lstmb = jnp.zeros((4, DP), _f32).at[:, :D].set(
        lstm_b.reshape(4, D)).reshape(1, 4 * DP)
    qWt = [_padT(q2e_W[i]) for i in range(NL)]
    qb = [_padb(q2e_b[i]) for i in range(NL)]
    hWt = [_padT(kb_head_W[i]) for i in range(NL)]
    hb = [_padb(kb_head_b[i]) for i in range(NL)]
    tWt = [_padT(kb_tail_W[i]) for i in range(NL)]
    tb = [_padb(kb_tail_b[i]) for i in range(NL)]
    sWt = [_padT(kb_self_W[i]) for i in range(NL)]
    sb = [_padb(kb_self_b[i]) for i in range(NL)]
    e1t = [_padT(e2e_W[i][:, 0 * D:1 * D]) for i in range(NL)]
    e2t = [_padT(e2e_W[i][:, 1 * D:2 * D]) for i in range(NL)]
    e3t = [_padT(e2e_W[i][:, 2 * D:3 * D]) for i in range(NL)]
    eb = [_padb(e2e_b[i]) for i in range(NL)]
    scw = jnp.zeros((1, DP), _f32).at[0, :D].set(score_W[0])
    scb = score_b.reshape(1, 1)

    le_pad = jnp.pad(local_entity, ((0, 0), (0, EP - E)), constant_values=NE)
    ent_gidx = le_pad.reshape(-1)
    word_idx = jnp.pad(query_text.T.reshape(-1), (0, 128),
                       constant_values=NW)
    rel_idx = jnp.pad(kb_fact_rel, ((0, 0), (0, FP - F)),
                      constant_values=NR).reshape(-1)
    head_pad = jnp.pad(head_idx, ((0, 0), (0, FP - F)))
    tail_pad = jnp.pad(tail_idx, ((0, 0), (0, FP - F)))
    boff = (jnp.arange(B, dtype=jnp.int32) * EP)[:, None]
    goff = ((jnp.arange(B, dtype=jnp.int32) % 4) * EP)[:, None]
    head_gidx = (head_pad + boff).reshape(-1)
    head_lidx = (head_pad + goff).reshape(-1)
    tail_lidx2d = (tail_pad + goff).reshape(-1).reshape(FB // 128, 128)
    pr0 = jnp.pad(q2e_adj_mat[:, :, 0], ((0, 0), (0, EP - E))).reshape(EB, 1)
    qmt = (query_text.T != NW).astype(_f32)
    le3 = le_pad.reshape(B, 1, EP)
    zeros512 = jnp.zeros((512, DP), _f32)

    ent_tab = _tc_proj(entity_table, entWt, entb, 4096)
    rel_tab = _tc_proj(relation_table, relWt, relb, 512)
    word_tab = _tc_pad(word_table, 8192)
    ent_emb = _sc_gather(ent_tab, ent_gidx, 128)
    word_rows = _sc_gather(word_tab, word_idx, 8)
    fact_emb = _sc_gather(rel_tab, rel_idx, 128)

    hs, node = _tc_lstm(word_rows, wih, whh, lstmb)
    hs_t = jnp.transpose(hs, (1, 0, 2))
    node3 = node.reshape(B, 1, DP)
    wt = _tc_att(hs_t, fact_emb, qmt.T.reshape(B, 1, Q))
    wt_flat = wt.reshape(-1)
    wt_col = wt.reshape(FB, 1)

    partials = _sc_scalar_scatter(head_lidx, wt_flat)
    hh, e2f = _tc_e0(ent_emb, partials, pr0, hWt[0], hb[0])
    pr = pr0

    for i in range(NL):
        g = _sc_gather(hh, head_gidx, 128)
        payload = _tc_payload(g, fact_emb, wt_col, sWt[i], sb[i],
                              tWt[i], tb[i])
        agg = _sc_scatter(zeros512, payload, tail_lidx2d)
        if i < NL - 1:
            ent_emb, pr, hh = _tc_combine(
                agg, ent_emb, pr, e2f, node3, qWt[i], qb[i], sWt[i], sb[i],
                e1t[i], e2t[i], e3t[i], eb[i], hWt[i + 1], hb[i + 1])
        else:
            score = _tc_final(
                agg, ent_emb, node3, le3, qWt[i], qb[i], sWt[i], sb[i],
                e1t[i], e2t[i], e3t[i], eb[i], scw, scb)

    return score.reshape(B, EP)[:, :E]


# speedup vs baseline: 8.6145x; 1.0722x over previous
import functools

import jax
import jax.numpy as jnp
from jax import lax
from jax.experimental import pallas as pl
from jax.experimental.pallas import tpu as pltpu
from jax.experimental.pallas import tpu_sc as plsc

B, E, F, Q, D = 8, 2000, 8000, 16, 100
EP, FP, DP = 2048, 8192, 128
EB, FB = B * EP, B * FP
NE, NR, NW = 200000, 500, 50000
NL = 3
NC, NS = 2, 16
VERY_NEG = -1e11
PL = 0.8
FS = 3.0
_f32 = jnp.float32


def _sc_gather(table, idx, CH):
    N = idx.shape[0]
    DW = table.shape[1]
    n = N // (NC * NS)
    nch = n // CH
    NB = min(nch, 4)
    mesh = plsc.VectorSubcoreMesh(core_axis_name="c", subcore_axis_name="s")

    @functools.partial(
        pl.kernel, mesh=mesh,
        out_type=jax.ShapeDtypeStruct((N, DW), _f32),
        scratch_types=[pltpu.VMEM((n,), jnp.int32)]
        + [pltpu.VMEM((CH, DW), _f32)] * NB
        + [pltpu.SemaphoreType.DMA] * (2 * NB))
    def k(tab, ix, out, idx_v, *rest):
        bufs = rest[:NB]
        gsems = rest[NB:2 * NB]
        osems = rest[2 * NB:]
        wid = lax.axis_index("s") * NC + lax.axis_index("c")
        base = wid * n
        pltpu.sync_copy(ix.at[pl.ds(base, n)], idx_v)

        def gath(j):
            return pltpu.async_copy(
                tab.at[idx_v.at[pl.ds(j * CH, CH)]], bufs[j % NB],
                gsems[j % NB])

        descs = [None] * nch
        couts = [None] * nch
        for j in range(NB):
            descs[j] = gath(j)
        for j in range(nch):
            descs[j].wait()
            couts[j] = pltpu.async_copy(
                bufs[j % NB], out.at[pl.ds(base + j * CH, CH)],
                osems[j % NB])
            if j + NB < nch:
                couts[j].wait()
                descs[j + NB] = gath(j + NB)
        for j in range(max(0, nch - NB), nch):
            if couts[j] is not None and j + NB >= nch:
                couts[j].wait()

    return k(table, idx)


def _sc_scatter(zeros, msgs, idx2d):
    EG, CH = 4 * EP, 128
    NCH = 16
    mesh = plsc.VectorSubcoreMesh(core_axis_name="c", subcore_axis_name="s")

    @functools.partial(
        pl.kernel, mesh=mesh,
        out_type=jax.ShapeDtypeStruct((2 * EG, DP), _f32),
        scratch_types=[pltpu.VMEM((CH,), jnp.int32),
                       pltpu.VMEM((CH,), jnp.int32),
                       pltpu.VMEM((CH, DP), _f32),
                       pltpu.VMEM((CH, DP), _f32),
                       pltpu.VMEM_SHARED((EG, DP), _f32),
                       pltpu.SemaphoreType.DMA,
                       pltpu.SemaphoreType.DMA,
                       pltpu.SemaphoreType.DMA])
    def k(zer, ms, ix2, out, ix0, ix1, bf0, bf1, acc, l0, l1, ssem):
        c = lax.axis_index("c")
        s = lax.axis_index("s")
        pltpu.sync_copy(zer, acc.at[pl.ds(s * 512, 512)])
        plsc.subcore_barrier()
        base = c * 256 + s * 16
        ixs = (ix0, ix1)
        bfs = (bf0, bf1)
        lsems = (l0, l1)

        def load(j):
            chunk = base + j
            d1 = pltpu.async_copy(ms.at[pl.ds(chunk * CH, CH)],
                                  bfs[j % 2], lsems[j % 2])
            d2 = pltpu.async_copy(ix2.at[chunk], ixs[j % 2], lsems[j % 2])
            return (d1, d2)

        descs = [None] * NCH
        descs[0] = load(0)
        scat = [None] * NCH
        for j in range(NCH):
            descs[j][0].wait()
            descs[j][1].wait()
            if j >= 1:
                scat[j - 1].wait()
            if j + 1 < NCH:
                descs[j + 1] = load(j + 1)
            scat[j] = pltpu.async_copy(bfs[j % 2], acc.at[ixs[j % 2]],
                                       ssem, add=True)
        scat[NCH - 1].wait()
        plsc.subcore_barrier()
        pltpu.sync_copy(acc.at[pl.ds(s * 512, 512)],
                        out.at[pl.ds(c * EG + s * 512, 512)])

    return k(zeros, msgs, idx2d)


def _sc_scalar_scatter(idx, val):
    EG, n = 4 * EP, FB // (NC * NS)
    mesh = plsc.VectorSubcoreMesh(core_axis_name="c", subcore_axis_name="s")

    @functools.partial(
        pl.kernel, mesh=mesh,
        compiler_params=pltpu.CompilerParams(needs_layout_passes=False),
        out_type=jax.ShapeDtypeStruct((2 * NS, EG), _f32),
        scratch_types=[pltpu.VMEM((n,), jnp.int32),
                       pltpu.VMEM((n,), _f32),
                       pltpu.VMEM((EG,), _f32)])
    def k(ix, vl, out, idx_v, val_v, acc):
        c = lax.axis_index("c")
        s = lax.axis_index("s")
        base = (c * NS + s) * n

        def zbody(i, carry):
            acc[pl.ds(i * 16, 16)] = jnp.zeros((16,), _f32)
            return carry

        lax.fori_loop(0, EG // 16, zbody, 0)
        pltpu.sync_copy(ix.at[pl.ds(base, n)], idx_v)
        pltpu.sync_copy(vl.at[pl.ds(base, n)], val_v)

        def body(j, carry):
            iv = idx_v[pl.ds(j * 16, 16)]
            vv = jnp.maximum(val_v[pl.ds(j * 16, 16)], 0.0)
            plsc.addupdate_scatter(acc, [iv], vv)
            return carry

        lax.fori_loop(0, n // 16, body, 0)
        pltpu.sync_copy(acc, out.at[c * NS + s])

    return k(idx, val)


def _dot(a, b):
    return jnp.dot(a, b, preferred_element_type=_f32)


def _sig(x):
    return 1.0 / (1.0 + jnp.exp(-x))


def _proj_body(x_ref, w_ref, b_ref, o_ref):
    o_ref[...] = _dot(x_ref[...], w_ref[...]) + b_ref[...]


def _pad_body(x_ref, o_ref):
    o_ref[...] = jnp.pad(x_ref[...], ((0, 0), (0, DP - D)))


def _tc_pad(x, bm):
    n = x.shape[0]
    return pl.pallas_call(
        _pad_body, grid=(pl.cdiv(n, bm),),
        in_specs=[pl.BlockSpec((bm, D), lambda i: (i, 0))],
        out_specs=pl.BlockSpec((bm, DP), lambda i: (i, 0)),
        out_shape=jax.ShapeDtypeStruct((n, DP), _f32))(x)


def _lstm_body(wr_ref, wih_ref, whh_ref, b_ref, hs_ref, node_ref):
    def step(q, carry):
        h, c = carry
        x = wr_ref[pl.ds(q * 8, 8), :]
        z = _dot(x, wih_ref[...]) + _dot(h, whh_ref[...]) + b_ref[...]
        i = _sig(z[:, 0:DP])
        f = _sig(z[:, DP:2 * DP])
        g = jnp.tanh(z[:, 2 * DP:3 * DP])
        o = _sig(z[:, 3 * DP:4 * DP])
        c = f * c + i * g
        h = o * jnp.tanh(c)
        hs_ref[pl.ds(q, 1)] = h[None]
        return (h, c)

    z0 = jnp.zeros((8, DP), _f32)
    h, _ = lax.fori_loop(0, Q, step, (z0, z0))
    node_ref[...] = h


def _onehot_fact(idx_ref, tab_ref):
    oh = (lax.broadcasted_iota(jnp.int32, (idx_ref.shape[0], 512), 1)
          == idx_ref[...]).astype(_f32)
    return _dot(oh, tab_ref[...])


def _att_body(hs_ref, ridx_ref, tab_ref, qmt_ref, o_ref):
    fact = _onehot_fact(ridx_ref, tab_ref)
    h = hs_ref[...].reshape(Q, DP)
    pen_row = qmt_ref[...].reshape(1, Q)
    pen = lax.dot_general(jnp.eye(Q, dtype=_f32), pen_row,
                          (((1,), (1,)), ((), ())),
                          preferred_element_type=_f32)
    sim = lax.dot_general(h, fact,
                          (((1,), (1,)), ((), ())),
                          preferred_element_type=_f32) * (1.0 / 10.0)
    masked = sim + (1.0 - pen) * VERY_NEG
    mx = jnp.max(masked, axis=0, keepdims=True)
    ex = jnp.exp(masked - mx)
    p = ex / jnp.sum(ex, axis=0, keepdims=True)
    wf = jnp.sum(p * sim, axis=0, keepdims=True)
    fmask = lax.broadcasted_iota(jnp.int32, (1, FP), 1) < F
    wmax = jnp.max(jnp.where(fmask, wf, -1e30))
    wt = jnp.exp(wf - wmax)
    o_ref[...] = jnp.where(fmask, wt, -1.0)[None]


def _e0_body(emb_ref, part_ref, pr_ref, w_ref, b_ref, hh_ref, e2f_ref):
    ones = jnp.ones((Q, 1), _f32)
    psum = lax.dot_general(part_ref[...], ones,
                           (((0,), (0,)), ((), ())),
                           preferred_element_type=_f32)
    e2f = jnp.maximum(psum, 1e-10)
    ratio = pr_ref[...] / e2f
    hh = _dot(emb_ref[...], w_ref[...]) + b_ref[...]
    colm = lax.broadcasted_iota(jnp.int32, (EP, DP), 1) == D
    hh_ref[...] = jnp.where(colm, ratio, hh)
    e2f_ref[...] = e2f


def _col(x):
    m = lax.broadcasted_iota(jnp.int32, x.shape, 1) == D
    return jnp.sum(jnp.where(m, x, 0.0), axis=1, keepdims=True)


def _payload_body(g_ref, ridx_ref, tab_ref, wt_ref, sw_ref, sb_ref,
                  tw_ref, tb_ref, o_ref):
    fact = _onehot_fact(ridx_ref, tab_ref)
    wt = wt_ref[...]
    valid = wt >= 0.0
    s = jnp.maximum(wt, 0.0) * _col(g_ref[...])
    e2f = jnp.maximum(_dot(fact, sw_ref[...]) + sb_ref[...]
                      + g_ref[...], 0.0)
    th = _dot(e2f, tw_ref[...])
    pay = jnp.where(valid, th * s + tb_ref[...], 0.0)
    colm = lax.broadcasted_iota(jnp.int32, (pay.shape[0], DP), 1) == D
    o_ref[...] = jnp.where(colm, s, pay)


def _combine_body(agg_ref, emb_ref, pr_ref, e2f_ref, node_ref, qw_ref,
                  qb_ref, sw_ref, sb_ref, w1_ref, w2_ref, w3_ref, eb_ref,
                  hw_ref, hb_ref, emb_o, pr_o, hh_o):
    nb = node_ref[...].reshape(1, DP)
    q2e = _dot(_dot(nb, qw_ref[...]) + qb_ref[...], w2_ref[...])
    f2e = jnp.maximum(_dot(emb_ref[...], sw_ref[...]) + sb_ref[...]
                      + agg_ref[...], 0.0)
    z = (_dot(emb_ref[...], w1_ref[...]) + q2e
         + _dot(f2e, w3_ref[...]) * FS + eb_ref[...])
    new_emb = jnp.maximum(z, 0.0)
    new_pr = PL * _col(agg_ref[...]) + (1.0 - PL) * pr_ref[...]
    hh = _dot(new_emb, hw_ref[...]) + hb_ref[...]
    ratio = new_pr / e2f_ref[...]
    colm = lax.broadcasted_iota(jnp.int32, (EP, DP), 1) == D
    emb_o[...] = new_emb
    pr_o[...] = new_pr
    hh_o[...] = jnp.where(colm, ratio, hh)


def _final_body(agg_ref, emb_ref, node_ref, le_ref, qw_ref, qb_ref, sw_ref,
                sb_ref, w1_ref, w2_ref, w3_ref, eb_ref, scw_ref, scb_ref,
                o_ref):
    nb = node_ref[...].reshape(1, DP)
    q2e = _dot(_dot(nb, qw_ref[...]) + qb_ref[...], w2_ref[...])
    f2e = jnp.maximum(_dot(emb_ref[...], sw_ref[...]) + sb_ref[...]
                      + agg_ref[...], 0.0)
    z = (_dot(emb_ref[...], w1_ref[...]) + q2e
         + _dot(f2e, w3_ref[...]) * FS + eb_ref[...])
    new_emb = jnp.maximum(z, 0.0)
    sc = lax.dot_general(scw_ref[...], new_emb,
                         (((1,), (1,)), ((), ())),
                         preferred_element_type=_f32) + scb_ref[...]
    mask = (le_ref[...].reshape(1, EP) != NE).astype(_f32)
    o_ref[...] = (sc + (1.0 - mask) * VERY_NEG)[None]


def _wspec(shape):
    nd = len(shape)
    return pl.BlockSpec(shape, lambda *a, _nd=nd: (0,) * _nd)


def _tc_proj(x, wt, b, bm):
    n, kk = x.shape
    return pl.pallas_call(
        _proj_body, grid=(pl.cdiv(n, bm),),
        in_specs=[pl.BlockSpec((bm, kk), lambda i: (i, 0)),
                  _wspec(wt.shape), _wspec(b.shape)],
        out_specs=pl.BlockSpec((bm, DP), lambda i: (i, 0)),
        out_shape=jax.ShapeDtypeStruct((n, DP), _f32))(x, wt, b)


def _tc_lstm(word_rows, wih, whh, b):
    return pl.pallas_call(
        _lstm_body,
        in_specs=[_wspec(word_rows.shape), _wspec(wih.shape),
                  _wspec(whh.shape), _wspec(b.shape)],
        out_specs=[_wspec((Q, 8, DP)), _wspec((8, DP))],
        out_shape=[jax.ShapeDtypeStruct((Q, 8, DP), _f32),
                   jax.ShapeDtypeStruct((8, DP), _f32)])(
                       word_rows, wih, whh, b)


def _tc_att(hs_t, rel_col, rel_tabp, qm3):
    return pl.pallas_call(
        _att_body, grid=(B,),
        in_specs=[pl.BlockSpec((1, Q, DP), lambda b: (b, 0, 0)),
                  pl.BlockSpec((FP, 1), lambda b: (b, 0)),
                  _wspec((512, DP)),
                  pl.BlockSpec((1, 1, Q), lambda b: (b, 0, 0))],
        out_specs=pl.BlockSpec((1, 1, FP), lambda b: (b, 0, 0)),
        out_shape=jax.ShapeDtypeStruct((B, 1, FP), _f32))(
            hs_t, rel_col, rel_tabp, qm3)


def _tc_e0(ent_emb, partials, pr0, hw, hb):
    return pl.pallas_call(
        _e0_body, grid=(B,),
        in_specs=[pl.BlockSpec((EP, DP), lambda b: (b, 0)),
                  pl.BlockSpec((NS, EP), lambda b: (b // 4, b % 4)),
                  pl.BlockSpec((EP, 1), lambda b: (b, 0)),
                  _wspec((DP, DP)), _wspec((1, DP))],
        out_specs=[pl.BlockSpec((EP, DP), lambda b: (b, 0)),
                   pl.BlockSpec((EP, 1), lambda b: (b, 0))],
        out_shape=[jax.ShapeDtypeStruct((EB, DP), _f32),
                   jax.ShapeDtypeStruct((EB, 1), _f32)])(
                       ent_emb, partials, pr0, hw, hb)


def _tc_payload(g, rel_col, rel_tabp, wt_col, sw, sb, tw, tb):
    bm = 4096
    return pl.pallas_call(
        _payload_body, grid=(FB // bm,),
        in_specs=[pl.BlockSpec((bm, DP), lambda i: (i, 0)),
                  pl.BlockSpec((bm, 1), lambda i: (i, 0)),
                  _wspec((512, DP)),
                  pl.BlockSpec((bm, 1), lambda i: (i, 0)),
                  _wspec((DP, DP)), _wspec((1, DP)),
                  _wspec((DP, DP)), _wspec((1, DP))],
        out_specs=pl.BlockSpec((bm, DP), lambda i: (i, 0)),
        out_shape=jax.ShapeDtypeStruct((FB, DP), _f32))(
            g, rel_col, rel_tabp, wt_col, sw, sb, tw, tb)


def _tc_combine(agg, ent_emb, pr, e2f, node, qw, qb, sw, sb, w1, w2, w3,
                eb, hw, hb):
    eblk = lambda b: (b, 0)
    return pl.pallas_call(
        _combine_body, grid=(B,),
        in_specs=[pl.BlockSpec((EP, DP), eblk), pl.BlockSpec((EP, DP), eblk),
                  pl.BlockSpec((EP, 1), eblk), pl.BlockSpec((EP, 1), eblk),
                  pl.BlockSpec((1, 1, DP), lambda b: (b, 0, 0)),
                  _wspec((DP, DP)), _wspec((1, DP)),
                  _wspec((DP, DP)), _wspec((1, DP)), _wspec((DP, DP)),
                  _wspec((DP, DP)), _wspec((DP, DP)), _wspec((1, DP)),
                  _wspec((DP, DP)), _wspec((1, DP))],
        out_specs=[pl.BlockSpec((EP, DP), eblk), pl.BlockSpec((EP, 1), eblk),
                   pl.BlockSpec((EP, DP), eblk)],
        out_shape=[jax.ShapeDtypeStruct((EB, DP), _f32),
                   jax.ShapeDtypeStruct((EB, 1), _f32),
                   jax.ShapeDtypeStruct((EB, DP), _f32)])(
                       agg, ent_emb, pr, e2f, node, qw, qb, sw, sb,
                       w1, w2, w3, eb, hw, hb)


def _tc_final(agg, ent_emb, node, le3, qw, qb, sw, sb, w1, w2, w3, eb,
              scw, scb):
    eblk = lambda b: (b, 0)
    return pl.pallas_call(
        _final_body, grid=(B,),
        in_specs=[pl.BlockSpec((EP, DP), eblk), pl.BlockSpec((EP, DP), eblk),
                  pl.BlockSpec((1, 1, DP), lambda b: (b, 0, 0)),
                  pl.BlockSpec((1, 1, EP), lambda b: (b, 0, 0)),
                  _wspec((DP, DP)), _wspec((1, DP)),
                  _wspec((DP, DP)), _wspec((1, DP)), _wspec((DP, DP)),
                  _wspec((DP, DP)), _wspec((DP, DP)), _wspec((1, DP)),
                  _wspec((1, DP)), _wspec((1, 1))],
        out_specs=pl.BlockSpec((1, 1, EP), lambda b: (b, 0, 0)),
        out_shape=jax.ShapeDtypeStruct((B, 1, EP), _f32))(
            agg, ent_emb, node, le3, qw, qb, sw, sb, w1, w2, w3, eb,
            scw, scb)


def _padT(w):
    o, i = w.shape
    return jnp.zeros((((i + 127) // 128) * 128, DP), _f32).at[:i, :o].set(w.T)


def _padTo(w):
    o, i = w.shape
    return jnp.zeros((i, DP), _f32).at[:, :o].set(w.T)


def _padb(b):
    return jnp.zeros((1, DP), _f32).at[0, :b.shape[0]].set(b)


def kernel(local_entity, q2e_adj_mat, kb_fact_rel, query_text, head_idx,
           tail_idx, entity_table, relation_table, word_table, ent_W, ent_b,
           rel_W, rel_b, lstm_Wih, lstm_Whh, lstm_b, q2e_W, q2e_b,
           kb_head_W, kb_head_b, kb_tail_W, kb_tail_b, kb_self_W, kb_self_b,
           e2e_W, e2e_b, score_W, score_b):
    local_entity = local_entity.astype(jnp.int32)
    kb_fact_rel = kb_fact_rel.astype(jnp.int32)
    query_text = query_text.astype(jnp.int32)
    head_idx = head_idx.astype(jnp.int32)
    tail_idx = tail_idx.astype(jnp.int32)

    entWt, entb = _padTo(ent_W), _padb(ent_b)
    relWt, relb = _padTo(rel_W), _padb(rel_b)
    wih = jnp.zeros((DP, 4, DP), _f32).at[:D, :, :D].set(
        lstm_Wih.reshape(4, D, D).transpose(2, 0, 1)).reshape(DP, 4 * DP)
    whh = jnp.zeros((DP, 4, DP), _f32).at[:D, :, :D].set(
        lstm_Whh.reshape(4, D, D).transpose(2, 0, 1)).reshape(DP, 4 * DP)
    lstmb = jnp.zeros((4, DP), _f32).at[:, :D].set(
        lstm_b.reshape(4, D)).reshape(1, 4 * DP)
    qWt = [_padT(q2e_W[i]) for i in range(NL)]
    qb = [_padb(q2e_b[i]) for i in range(NL)]
    hWt = [_padT(kb_head_W[i]) for i in range(NL)]
    hb = [_padb(kb_head_b[i]) for i in range(NL)]
    tWt = [_padT(kb_tail_W[i]) for i in range(NL)]
    tb = [_padb(kb_tail_b[i]) for i in range(NL)]
    sWt = [_padT(kb_self_W[i]) for i in range(NL)]
    sb = [_padb(kb_self_b[i]) for i in range(NL)]
    e1t = [_padT(e2e_W[i][:, 0 * D:1 * D]) for i in range(NL)]
    e2t = [_padT(e2e_W[i][:, 1 * D:2 * D]) for i in range(NL)]
    e3t = [_padT(e2e_W[i][:, 2 * D:3 * D]) for i in range(NL)]
    eb = [_padb(e2e_b[i]) for i in range(NL)]
    scw = jnp.zeros((1, DP), _f32).at[0, :D].set(score_W[0])
    scb = score_b.reshape(1, 1)

    le_pad = jnp.pad(local_entity, ((0, 0), (0, EP - E)), constant_values=NE)
    ent_gidx = le_pad.reshape(-1)
    word_idx = jnp.pad(query_text.T.reshape(-1), (0, 128),
                       constant_values=NW)
    rel_idx = jnp.pad(kb_fact_rel, ((0, 0), (0, FP - F)),
                      constant_values=NR).reshape(-1)
    head_pad = jnp.pad(head_idx, ((0, 0), (0, FP - F)))
    tail_pad = jnp.pad(tail_idx, ((0, 0), (0, FP - F)))
    boff = (jnp.arange(B, dtype=jnp.int32) * EP)[:, None]
    goff = ((jnp.arange(B, dtype=jnp.int32) % 4) * EP)[:, None]
    head_gidx = (head_pad + boff).reshape(-1)
    head_lidx = (head_pad + goff).reshape(-1)
    tail_lidx2d = (tail_pad + goff).reshape(-1).reshape(FB // 128, 128)
    pr0 = jnp.pad(q2e_adj_mat[:, :, 0], ((0, 0), (0, EP - E))).reshape(EB, 1)
    qmt = (query_text.T != NW).astype(_f32)
    le3 = le_pad.reshape(B, 1, EP)
    zeros512 = jnp.zeros((512, DP), _f32)

    ent_tab = _tc_proj(entity_table, entWt, entb, 4096)
    rel_tab = _tc_proj(relation_table, relWt, relb, 512)
    rel_tabp = jnp.pad(rel_tab, ((0, 11), (0, 0)))
    word_tab = _tc_pad(word_table, 8192)
    ent_emb = _sc_gather(ent_tab, ent_gidx, 128)
    word_rows = _sc_gather(word_tab, word_idx, 8)
    rel_col = rel_idx.reshape(FB, 1)

    hs, node = _tc_lstm(word_rows, wih, whh, lstmb)
    hs_t = jnp.transpose(hs, (1, 0, 2))
    node3 = node.reshape(B, 1, DP)
    wt = _tc_att(hs_t, rel_col, rel_tabp, qmt.T.reshape(B, 1, Q))
    wt_flat = wt.reshape(-1)
    wt_col = wt.reshape(FB, 1)

    partials = _sc_scalar_scatter(head_lidx, wt_flat)
    hh, e2f = _tc_e0(ent_emb, partials, pr0, hWt[0], hb[0])
    pr = pr0

    for i in range(NL):
        g = _sc_gather(hh, head_gidx, 128)
        payload = _tc_payload(g, rel_col, rel_tabp, wt_col, sWt[i], sb[i],
                              tWt[i], tb[i])
        agg = _sc_scatter(zeros512, payload, tail_lidx2d)
        if i < NL - 1:
            ent_emb, pr, hh = _tc_combine(
                agg, ent_emb, pr, e2f, node3, qWt[i], qb[i], sWt[i], sb[i],
                e1t[i], e2t[i], e3t[i], eb[i], hWt[i + 1], hb[i + 1])
        else:
            score = _tc_final(
                agg, ent_emb, node3, le3, qWt[i], qb[i], sWt[i], sb[i],
                e1t[i], e2t[i], e3t[i], eb[i], scw, scb)

    return score.reshape(B, EP)[:, :E]
